# baseline (speedup 1.0000x reference)
import math
import jax
import jax.numpy as jnp
import numpy as np
from jax.experimental import pallas as pl

_SIGMAS = np.array([0.02, 0.02, 0.05, 0.05, 0.05], dtype=np.float32)


def _proj_matrix(d):
    a = np.triu(np.ones((d, d), dtype=np.float32), 1) - np.diag(np.arange(1, d + 1).astype(np.float32))
    a = np.concatenate([np.ones((1, d), dtype=np.float32), a], axis=0)
    b = np.diag((1.0 / np.sqrt((np.arange(1, d + 1) * np.arange(2, d + 2)).astype(np.float32))))
    return (a @ b).astype(np.float32)


def _canon_simplex(d):
    rows = [[i] * (d + 1 - i) + [-(d + 1 - i)] * i for i in range(d + 1)]
    return np.array(rows, dtype=np.int64).T


def _basis(d):
    ed = d + 1
    return (ed * np.eye(ed) - np.ones((ed, ed))).astype(np.int64)


def _coords(x, sigmas):
    n, d = x.shape
    ed = d + 1
    sc = x / jnp.asarray(sigmas).reshape(1, d)
    sc = sc / (math.sqrt(2.0 / 3.0) * ed)
    e = jnp.asarray(_proj_matrix(d))
    p = sc @ e.T
    l0 = jnp.floor(p / ed) * ed
    residual = p - l0
    indices = jnp.argsort(-residual, axis=1)
    ranks = jnp.argsort(indices, axis=1).astype(p.dtype)
    greedy = ranks + l0.sum(axis=1, keepdims=True) / ed
    l0 = jnp.where(greedy < 0, l0 + ed, jnp.where(greedy > d, l0 - ed, l0))
    ranks = jnp.where(greedy < 0, greedy + ed, jnp.where(greedy > d, greedy - ed, greedy))
    return p, l0, ranks


def _bary(x, sigmas):
    n, d = x.shape
    ed = d + 1
    p, l0, ranks = _coords(x, sigmas)
    residual = (p - l0) / ed
    order = jnp.argsort(-ranks, axis=1)
    g = jnp.take_along_axis(residual, order, axis=1)
    b = jnp.diff(g, axis=1)
    b = jnp.concatenate([1.0 - b.sum(axis=1, keepdims=True), b], axis=1)
    return b


def _pack(pts):
    s = pts.astype(jnp.int32) + 512
    ed = s.shape[-1]
    h = ed // 2
    k1 = s[..., 0]
    for j in range(1, h):
        k1 = k1 * 1024 + s[..., j]
    k2 = s[..., h]
    for j in range(h + 1, ed):
        k2 = k2 * 1024 + s[..., j]
    return k1, k2


def _lookup(uk1, uk2, qk1, qk2):
    mm = uk1.shape[0]
    lo = jnp.zeros(qk1.shape, dtype=jnp.int32)
    hi = jnp.full(qk1.shape, mm, dtype=jnp.int32)
    for _ in range(int(math.ceil(math.log2(mm))) + 1):
        mid = (lo + hi) // 2
        mk1 = uk1[mid]
        mk2 = uk2[mid]
        less = (mk1 < qk1) | ((mk1 == qk1) & (mk2 < qk2))
        lo = jnp.where(less, mid + 1, lo)
        hi = jnp.where(less, hi, mid)
    fk1 = uk1[jnp.minimum(lo, mm - 1)]
    fk2 = uk2[jnp.minimum(lo, mm - 1)]
    found = (lo < mm) & (fk1 == qk1) & (fk2 == qk2)
    return jnp.where(found, lo, -1)


def _fit(x, sigmas):
    n, d = x.shape
    ed = d + 1
    m = n * ed
    _, l0, ranks = _coords(x, sigmas)
    l0 = l0.astype(jnp.int32)
    ri = ranks.astype(jnp.int32)
    cs = jnp.asarray(_canon_simplex(d).astype(np.int32))
    pts = l0[:, None, :] + jnp.take(cs, ri, axis=1).transpose(1, 0, 2)
    pts_flat = pts.reshape(-1, ed)
    k1, k2 = _pack(pts_flat)
    perm = jnp.lexsort((k2, k1))
    sk1 = k1[perm]
    sk2 = k2[perm]
    new = jnp.concatenate([jnp.ones((1,), dtype=bool),
                           (sk1[1:] != sk1[:-1]) | (sk2[1:] != sk2[:-1])])
    ids_sorted = jnp.cumsum(new.astype(jnp.int32)) - 1
    inv = jnp.zeros((m,), dtype=jnp.int32).at[perm].set(ids_sorted)
    simplices = inv.reshape(n, ed)
    slot = jnp.where(new, ids_sorted, m)
    sentinel = jnp.iinfo(jnp.int32).max
    uk1 = jnp.full((m,), sentinel, dtype=jnp.int32).at[slot].set(sk1, mode='drop')
    uk2 = jnp.full((m,), sentinel, dtype=jnp.int32).at[slot].set(sk2, mode='drop')
    uniq = jnp.zeros((m, ed), dtype=jnp.int32).at[slot].set(pts_flat[perm], mode='drop')
    off = jnp.asarray(_basis(d).astype(np.int32))
    cand = jnp.stack([uniq[:, None, :] + off[None], uniq[:, None, :] - off[None]], axis=1)
    qk1, qk2 = _pack(cand.reshape(-1, ed))
    neighbors = _lookup(uk1, uk2, qk1, qk2).reshape(m, 2, ed)
    return simplices, neighbors, m


def _filter1(y, b, simplices, neighbors, m, d):
    n, c = y.shape
    ed = d + 1
    yb = b[:, :, None] * y[:, None, :]
    s = jnp.zeros((m, c), dtype=y.dtype).at[simplices.reshape(-1)].add(yb.reshape(-1, c))
    yc = jnp.concatenate([jnp.zeros((1, c), dtype=y.dtype), s], axis=0)
    for dd in range(ed):
        idx = (neighbors[:, :, dd] + 1).reshape(-1)
        yc = yc.at[1:].add(yc[idx].reshape(m, 2, c).mean(axis=1))
    out = yc[simplices.reshape(-1) + 1].reshape(n, ed, c)
    out = jnp.einsum('bij,bi->bj', out, b)
    alpha = 1.0 / (1.0 + 2.0 ** (-d))
    return out * alpha


def kernel(x, y):
    n, d = x.shape
    simplices, neighbors, m = _fit(x, _SIGMAS)
    b = _bary(x, _SIGMAS)
    ones = jnp.ones((n, 1), dtype=x.dtype)
    norms = 1.0 / jnp.sqrt(_filter1(ones, b, simplices, neighbors, m, d) + 1e-20)
    out = _filter1(y * norms, b, simplices, neighbors, m, d) * norms
    return out


# speedup vs baseline: 11.3159x; 11.3159x over previous
import functools
import math
import jax
import jax.numpy as jnp
import numpy as np
from jax import lax
from jax.experimental import pallas as pl
from jax.experimental.pallas import tpu as pltpu
from jax.experimental.pallas import tpu_sc as plsc

_SIGMAS = np.array([0.02, 0.02, 0.05, 0.05, 0.05], dtype=np.float32)

_N = 65536
_D = 5
_ED = 6
_C = 16
_M = _N * _ED
_PAD = 8
_ROWS = _M + _PAD
_NC, _NS = 2, 16
_NW = _NC * _NS
_CH = _M // _NW
_GB = 128
_VWIN = 1024
_PCH = _N // _NW
_PW = 256
_ALPHA = 1.0 / (1.0 + 2.0 ** (-_D))

_MESH = plsc.VectorSubcoreMesh(core_axis_name="c", subcore_axis_name="s")


@functools.partial(
    pl.kernel,
    out_type=jax.ShapeDtypeStruct((_ROWS, _C), jnp.float32),
    mesh=_MESH,
    compiler_params=pltpu.CompilerParams(use_tc_tiling_on_sc=False),
    scratch_types=[
        pltpu.VMEM((2 * _CH,), jnp.int32),
        pltpu.VMEM((2 * _VWIN, _C), jnp.float32),
        pltpu.VMEM((_VWIN, _C), jnp.float32),
        pltpu.VMEM((_VWIN, _C), jnp.float32),
        pltpu.SemaphoreType.DMA,
    ],
)
def _blur_step(yc_in, nbi, yc_out, idx_all, gath, old, outb, sem):
    wid = lax.axis_index("s") * _NC + lax.axis_index("c")
    base = wid * _CH
    pltpu.sync_copy(nbi.at[pl.ds(2 * base, 2 * _CH)], idx_all)

    @pl.when(wid == 0)
    def _zero_rows():
        for r in range(_PAD):
            outb[r, :] = jnp.zeros((_C,), jnp.float32)
        pltpu.sync_copy(outb.at[pl.ds(0, _PAD)], yc_out.at[pl.ds(0, _PAD)])

    def win_body(w, carry):
        descs = [
            pltpu.async_copy(
                yc_in.at[idx_all.at[pl.ds(w * (2 * _VWIN) + j * _GB, _GB)]],
                gath.at[pl.ds(j * _GB, _GB)],
                sem,
            )
            for j in range(2 * _VWIN // _GB)
        ]
        pltpu.sync_copy(yc_in.at[pl.ds(_PAD + base + w * _VWIN, _VWIN)], old)
        for dsc in descs:
            dsc.wait()

        def row_body(i, c2):
            g0 = gath[2 * i, :]
            g1 = gath[2 * i + 1, :]
            outb[i, :] = old[i, :] + 0.5 * (g0 + g1)
            return c2

        lax.fori_loop(0, _VWIN, row_body, 0)
        pltpu.sync_copy(outb, yc_out.at[pl.ds(_PAD + base + w * _VWIN, _VWIN)])
        return carry

    lax.fori_loop(0, _CH // _VWIN, win_body, 0)


@functools.partial(
    pl.kernel,
    out_type=jax.ShapeDtypeStruct((_N, _C), jnp.float32),
    mesh=_MESH,
    compiler_params=pltpu.CompilerParams(use_tc_tiling_on_sc=False),
    scratch_types=[
        pltpu.VMEM((_ED * _PCH,), jnp.int32),
        pltpu.VMEM((_ED * _PW, _C), jnp.float32),
        pltpu.VMEM((_PW, _C), jnp.float32),
        pltpu.VMEM((_PW, _C), jnp.float32),
        pltpu.SemaphoreType.DMA,
    ],
)
def _slice_step(yc, sims, bpad, out_hbm, idxs, rows, bww, outb, sem):
    wid = lax.axis_index("s") * _NC + lax.axis_index("c")
    pb = wid * _PCH
    pltpu.sync_copy(sims.at[pl.ds(_ED * pb, _ED * _PCH)], idxs)

    def win_body(w, carry):
        descs = [
            pltpu.async_copy(
                yc.at[idxs.at[pl.ds(w * (_ED * _PW) + j * _GB, _GB)]],
                rows.at[pl.ds(j * _GB, _GB)],
                sem,
            )
            for j in range(_ED * _PW // _GB)
        ]
        pltpu.sync_copy(bpad.at[pl.ds(pb + w * _PW, _PW)], bww)
        for dsc in descs:
            dsc.wait()

        def pt_body(i, c2):
            bv = bww[i, :]
            acc = bv[0] * rows[_ED * i, :]
            for j in range(1, _ED):
                acc = acc + bv[j] * rows[_ED * i + j, :]
            outb[i, :] = acc * _ALPHA
            return c2

        lax.fori_loop(0, _PW, pt_body, 0)
        pltpu.sync_copy(outb, out_hbm.at[pl.ds(pb + w * _PW, _PW)])
        return carry

    lax.fori_loop(0, _PCH // _PW, win_body, 0)


_Q = _M * 2 * _ED
_QCH = _Q // _NW
_QW = 2048
_NSTEP = 20
_PAD1 = 16
_ROWS1 = _M + _PAD1


@functools.partial(
    pl.kernel,
    out_type=jax.ShapeDtypeStruct((_Q,), jnp.int32),
    mesh=_MESH,
    compiler_params=pltpu.CompilerParams(use_tc_tiling_on_sc=False),
    scratch_types=[
        pltpu.VMEM((_QW,), jnp.int32),
        pltpu.VMEM((_QW,), jnp.int32),
        pltpu.VMEM((_QW,), jnp.int32),
        pltpu.VMEM((_QW,), jnp.int32),
        pltpu.VMEM((_QW,), jnp.int32),
        pltpu.VMEM((_QW,), jnp.int32),
        pltpu.VMEM((_QW,), jnp.int32),
        pltpu.SemaphoreType.DMA,
    ],
)
def _lookup_sc(uk1, uk2, qk1, qk2, out, q1, q2, lo, hi, mid, mk1, mk2, sem):
    wid = lax.axis_index("s") * _NC + lax.axis_index("c")
    qb = wid * _QCH
    nblk = _QW // 16
    ndma = _QW // _GB

    def win_body(w, carry):
        o = qb + w * _QW
        pltpu.sync_copy(qk1.at[pl.ds(o, _QW)], q1)
        pltpu.sync_copy(qk2.at[pl.ds(o, _QW)], q2)

        def init_blk(i, c):
            s = pl.ds(i * 16, 16)
            lo[s] = jnp.zeros((16,), jnp.int32)
            hi[s] = jnp.full((16,), _M, jnp.int32)
            return c

        lax.fori_loop(0, nblk, init_blk, 0)

        def gather_mid():
            def fire(j, c):
                sl = pl.ds(j * _GB, _GB)
                pltpu.async_copy(uk1.at[mid.at[sl]], mk1.at[sl], sem)
                pltpu.async_copy(uk2.at[mid.at[sl]], mk2.at[sl], sem)
                return c

            lax.fori_loop(0, ndma, fire, 0)
            pltpu.make_async_copy(uk1.at[pl.ds(0, _QW)], mk1, sem).wait()
            pltpu.make_async_copy(uk2.at[pl.ds(0, _QW)], mk2, sem).wait()

        for _step in range(_NSTEP):
            def mid_blk(i, c):
                s = pl.ds(i * 16, 16)
                mid[s] = lax.shift_right_arithmetic(lo[s] + hi[s], 1)
                return c

            lax.fori_loop(0, nblk, mid_blk, 0)
            gather_mid()

            def upd_blk(i, c):
                s = pl.ds(i * 16, 16)
                m1 = mk1[s]
                m2 = mk2[s]
                less = (m1 < q1[s]) | ((m1 == q1[s]) & (m2 < q2[s]))
                mm = mid[s]
                lo[s] = jnp.where(less, mm + 1, lo[s])
                hi[s] = jnp.where(less, hi[s], mm)
                return c

            lax.fori_loop(0, nblk, upd_blk, 0)

        def clamp_blk(i, c):
            s = pl.ds(i * 16, 16)
            mid[s] = jnp.minimum(lo[s], _M - 1)
            return c

        lax.fori_loop(0, nblk, clamp_blk, 0)
        gather_mid()

        def res_blk(i, c):
            s = pl.ds(i * 16, 16)
            found = (lo[s] < _M) & (mk1[s] == q1[s]) & (mk2[s] == q2[s])
            mid[s] = jnp.where(found, lo[s], -1)
            return c

        lax.fori_loop(0, nblk, res_blk, 0)
        pltpu.sync_copy(mid, out.at[pl.ds(o, _QW)])
        return carry

    lax.fori_loop(0, _QCH // _QW, win_body, 0)


@functools.partial(
    pl.kernel,
    out_type=jax.ShapeDtypeStruct((_ROWS1,), jnp.float32),
    mesh=_MESH,
    compiler_params=pltpu.CompilerParams(use_tc_tiling_on_sc=False),
    scratch_types=[
        pltpu.VMEM((_CH,), jnp.int32),
        pltpu.VMEM((_CH,), jnp.int32),
        pltpu.VMEM((_CH,), jnp.float32),
        pltpu.VMEM((_CH,), jnp.float32),
        pltpu.VMEM((_CH,), jnp.float32),
        pltpu.VMEM((_CH,), jnp.float32),
        pltpu.SemaphoreType.DMA,
    ],
)
def _blur1_step(w_in, nbp, nbm, w_out, ixp, ixm, gp, gm, old, ob, sem):
    wid = lax.axis_index("s") * _NC + lax.axis_index("c")
    base = wid * _CH
    pltpu.sync_copy(nbp.at[pl.ds(base, _CH)], ixp)
    pltpu.sync_copy(nbm.at[pl.ds(base, _CH)], ixm)

    @pl.when(wid == 0)
    def _zero_pad():
        ob[pl.ds(0, 16)] = jnp.zeros((16,), jnp.float32)
        pltpu.sync_copy(ob.at[pl.ds(0, 16)], w_out.at[pl.ds(0, _PAD1)])

    def fire(j, c):
        sl = pl.ds(j * _GB, _GB)
        pltpu.async_copy(w_in.at[ixp.at[sl]], gp.at[sl], sem)
        pltpu.async_copy(w_in.at[ixm.at[sl]], gm.at[sl], sem)
        return c

    lax.fori_loop(0, _CH // _GB, fire, 0)
    pltpu.sync_copy(w_in.at[pl.ds(_PAD1 + base, _CH)], old)
    pltpu.make_async_copy(w_in.at[pl.ds(0, _CH)], gp, sem).wait()
    pltpu.make_async_copy(w_in.at[pl.ds(0, _CH)], gm, sem).wait()

    def blk(i, c):
        s = pl.ds(i * 16, 16)
        ob[s] = old[s] + 0.5 * (gp[s] + gm[s])
        return c

    lax.fori_loop(0, _CH // 16, blk, 0)
    pltpu.sync_copy(ob, w_out.at[pl.ds(_PAD1 + base, _CH)])


@functools.partial(
    pl.kernel,
    out_type=jax.ShapeDtypeStruct((_M,), jnp.float32),
    mesh=_MESH,
    compiler_params=pltpu.CompilerParams(use_tc_tiling_on_sc=False),
    scratch_types=[
        pltpu.VMEM((_CH,), jnp.int32),
        pltpu.VMEM((_CH,), jnp.float32),
        pltpu.SemaphoreType.DMA,
    ],
)
def _gather1(w_in, idx, out, ix, g, sem):
    wid = lax.axis_index("s") * _NC + lax.axis_index("c")
    base = wid * _CH
    pltpu.sync_copy(idx.at[pl.ds(base, _CH)], ix)

    def fire(j, c):
        sl = pl.ds(j * _GB, _GB)
        pltpu.async_copy(w_in.at[ix.at[sl]], g.at[sl], sem)
        return c

    lax.fori_loop(0, _CH // _GB, fire, 0)
    pltpu.make_async_copy(w_in.at[pl.ds(0, _CH)], g, sem).wait()
    pltpu.sync_copy(g, out.at[pl.ds(base, _CH)])


def _proj_matrix(d):
    a = np.triu(np.ones((d, d), dtype=np.float32), 1) - np.diag(np.arange(1, d + 1).astype(np.float32))
    a = np.concatenate([np.ones((1, d), dtype=np.float32), a], axis=0)
    b = np.diag((1.0 / np.sqrt((np.arange(1, d + 1) * np.arange(2, d + 2)).astype(np.float32))))
    return (a @ b).astype(np.float32)


def _canon_simplex(d):
    rows = [[i] * (d + 1 - i) + [-(d + 1 - i)] * i for i in range(d + 1)]
    return np.array(rows, dtype=np.int64).T


def _basis(d):
    ed = d + 1
    return (ed * np.eye(ed) - np.ones((ed, ed))).astype(np.int64)


def _coords(x, sigmas):
    n, d = x.shape
    ed = d + 1
    sc = x / jnp.asarray(sigmas).reshape(1, d)
    sc = sc / (math.sqrt(2.0 / 3.0) * ed)
    e = jnp.asarray(_proj_matrix(d))
    p = sc @ e.T
    l0 = jnp.floor(p / ed) * ed
    residual = p - l0
    indices = jnp.argsort(-residual, axis=1)
    ranks = jnp.argsort(indices, axis=1).astype(p.dtype)
    greedy = ranks + l0.sum(axis=1, keepdims=True) / ed
    l0 = jnp.where(greedy < 0, l0 + ed, jnp.where(greedy > d, l0 - ed, l0))
    ranks = jnp.where(greedy < 0, greedy + ed, jnp.where(greedy > d, greedy - ed, greedy))
    return p, l0, ranks


def _bary(x, sigmas):
    n, d = x.shape
    ed = d + 1
    p, l0, ranks = _coords(x, sigmas)
    residual = (p - l0) / ed
    order = jnp.argsort(-ranks, axis=1)
    g = jnp.take_along_axis(residual, order, axis=1)
    b = jnp.diff(g, axis=1)
    b = jnp.concatenate([1.0 - b.sum(axis=1, keepdims=True), b], axis=1)
    return b


def _pack(pts):
    s = pts.astype(jnp.int32) + 512
    ed = s.shape[-1]
    h = ed // 2
    k1 = s[..., 0]
    for j in range(1, h):
        k1 = k1 * 1024 + s[..., j]
    k2 = s[..., h]
    for j in range(h + 1, ed):
        k2 = k2 * 1024 + s[..., j]
    return k1, k2


def _lookup(uk1, uk2, qk1, qk2):
    mm = uk1.shape[0]
    lo = jnp.zeros(qk1.shape, dtype=jnp.int32)
    hi = jnp.full(qk1.shape, mm, dtype=jnp.int32)
    for _ in range(int(math.ceil(math.log2(mm))) + 1):
        mid = (lo + hi) // 2
        mk1 = uk1[mid]
        mk2 = uk2[mid]
        less = (mk1 < qk1) | ((mk1 == qk1) & (mk2 < qk2))
        lo = jnp.where(less, mid + 1, lo)
        hi = jnp.where(less, hi, mid)
    fk1 = uk1[jnp.minimum(lo, mm - 1)]
    fk2 = uk2[jnp.minimum(lo, mm - 1)]
    found = (lo < mm) & (fk1 == qk1) & (fk2 == qk2)
    return jnp.where(found, lo, -1)


def _fit(x, sigmas):
    n, d = x.shape
    ed = d + 1
    m = n * ed
    _, l0, ranks = _coords(x, sigmas)
    l0 = l0.astype(jnp.int32)
    ri = ranks.astype(jnp.int32)
    cs = jnp.asarray(_canon_simplex(d).astype(np.int32))
    pts = l0[:, None, :] + jnp.take(cs, ri, axis=1).transpose(1, 0, 2)
    pts_flat = pts.reshape(-1, ed)
    k1, k2 = _pack(pts_flat)
    perm = jnp.lexsort((k2, k1))
    sk1 = k1[perm]
    sk2 = k2[perm]
    new = jnp.concatenate([jnp.ones((1,), dtype=bool),
                           (sk1[1:] != sk1[:-1]) | (sk2[1:] != sk2[:-1])])
    ids_sorted = jnp.cumsum(new.astype(jnp.int32)) - 1
    inv = jnp.zeros((m,), dtype=jnp.int32).at[perm].set(ids_sorted)
    simplices = inv.reshape(n, ed)
    slot = jnp.where(new, ids_sorted, m)
    sentinel = jnp.iinfo(jnp.int32).max
    uk1 = jnp.full((m,), sentinel, dtype=jnp.int32).at[slot].set(sk1, mode='drop')
    uk2 = jnp.full((m,), sentinel, dtype=jnp.int32).at[slot].set(sk2, mode='drop')
    uniq = jnp.zeros((m, ed), dtype=jnp.int32).at[slot].set(pts_flat[perm], mode='drop')
    off = jnp.asarray(_basis(d).astype(np.int32))
    cand = jnp.stack([uniq[:, None, :] + off[None], uniq[:, None, :] - off[None]], axis=1)
    qk1, qk2 = _pack(cand.reshape(-1, ed))
    neighbors = _lookup_sc(uk1, uk2, qk1, qk2).reshape(m, 2, ed)
    return simplices, neighbors, m


def _filter_sc(yq, b, sims_flat, simsp1, nbis, bpad):
    yb = b[:, :, None] * yq[:, None, :]
    s = jnp.zeros((_M, _C), dtype=jnp.float32).at[sims_flat].add(yb.reshape(-1, _C))
    yc = jnp.concatenate([jnp.zeros((_PAD, _C), dtype=jnp.float32), s], axis=0)
    for dd in range(_ED):
        yc = _blur_step(yc, nbis[dd])
    return _slice_step(yc, simsp1, bpad)


def kernel(x, y):
    simplices, neighbors, m = _fit(x, _SIGMAS)
    b = _bary(x, _SIGMAS)
    sims_flat = simplices.reshape(-1)
    simsp1 = sims_flat + _PAD
    nbis = [(neighbors[:, :, dd] + _PAD).reshape(-1) for dd in range(_ED)]
    bpad = jnp.zeros((_N, _C), dtype=jnp.float32).at[:, :_ED].set(b)

    b_flat = b.reshape(-1)
    s1 = jnp.zeros((_M,), dtype=jnp.float32).at[sims_flat].add(b_flat)
    w = jnp.concatenate([jnp.zeros((_PAD1,), jnp.float32), s1])
    nbp = [neighbors[:, 0, dd] + _PAD1 for dd in range(_ED)]
    nbm = [neighbors[:, 1, dd] + _PAD1 for dd in range(_ED)]
    for dd in range(_ED):
        w = _blur1_step(w, nbp[dd], nbm[dd])
    g1 = _gather1(w, sims_flat + _PAD1)
    r1 = (b_flat * g1).reshape(_N, _ED).sum(axis=1) * _ALPHA
    norms = (1.0 / jnp.sqrt(r1 + 1e-20))[:, None]

    out = _filter_sc(y * norms, b, sims_flat, simsp1, nbis, bpad)
    return out * norms


# speedup vs baseline: 321.8457x; 28.4418x over previous
import functools
import math
import jax
import jax.numpy as jnp
import numpy as np
from jax import lax
from jax.experimental import pallas as pl
from jax.experimental.pallas import tpu as pltpu
from jax.experimental.pallas import tpu_sc as plsc

_SIGMAS = np.array([0.02, 0.02, 0.05, 0.05, 0.05], dtype=np.float32)

_N = 65536
_D = 5
_ED = 6
_C = 16
_M = _N * _ED
_PAD = 8
_ROWS = _M + _PAD
_NC, _NS = 2, 16
_NW = _NC * _NS
_CH = _M // _NW
_GB = 128
_BRR = 512
_VWIN = 1024
_PCH = _N // _NW
_PW = 256
_ALPHA = 1.0 / (1.0 + 2.0 ** (-_D))

_MESH = plsc.VectorSubcoreMesh(core_axis_name="c", subcore_axis_name="s")


@functools.partial(
    pl.kernel,
    out_type=jax.ShapeDtypeStruct((_ROWS, _C), jnp.float32),
    mesh=_MESH,
    compiler_params=pltpu.CompilerParams(use_tc_tiling_on_sc=False),
    scratch_types=[
        pltpu.VMEM((2 * _BRR,), jnp.int32),
        pltpu.VMEM((2 * _BRR, _C), jnp.float32),
        pltpu.VMEM((_BRR, _C), jnp.float32),
        pltpu.VMEM((_BRR, _C), jnp.float32),
        pltpu.VMEM((16,), jnp.int32),
        pltpu.SemaphoreType.DMA,
    ],
)
def _blur_step(yc_in, nbi, uarr, yc_out, idx_all, gath, old, outb, ub, sem):
    wid = lax.axis_index("s") * _NC + lax.axis_index("c")
    pltpu.sync_copy(uarr, ub)
    u = ub[pl.ds(0, 16)][0]

    @pl.when(wid == 0)
    def _zero_rows():
        for r in range(_PAD):
            outb[r, :] = jnp.zeros((_C,), jnp.float32)
        pltpu.sync_copy(outb.at[pl.ds(0, _PAD)], yc_out.at[pl.ds(0, _PAD)])

    def blk_body(t, carry):
        row0 = (wid + _NW * t) * _BRR

        @pl.when(row0 < u)
        def _go():
            pltpu.sync_copy(nbi.at[pl.ds(2 * row0, 2 * _BRR)], idx_all)
            descs = [
                pltpu.async_copy(
                    yc_in.at[idx_all.at[pl.ds(j * _GB, _GB)]],
                    gath.at[pl.ds(j * _GB, _GB)],
                    sem,
                )
                for j in range(2 * _BRR // _GB)
            ]
            pltpu.sync_copy(yc_in.at[pl.ds(_PAD + row0, _BRR)], old)
            for dsc in descs:
                dsc.wait()

            def row_body(i, c2):
                g0 = gath[2 * i, :]
                g1 = gath[2 * i + 1, :]
                outb[i, :] = old[i, :] + 0.5 * (g0 + g1)
                return c2

            lax.fori_loop(0, _BRR, row_body, 0)
            pltpu.sync_copy(outb.at[pl.ds(0, _BRR)], yc_out.at[pl.ds(_PAD + row0, _BRR)])

        return carry

    lax.fori_loop(0, _CH // _BRR, blk_body, 0)


@functools.partial(
    pl.kernel,
    out_type=jax.ShapeDtypeStruct((_N, _C), jnp.float32),
    mesh=_MESH,
    compiler_params=pltpu.CompilerParams(use_tc_tiling_on_sc=False),
    scratch_types=[
        pltpu.VMEM((_ED * _PCH,), jnp.int32),
        pltpu.VMEM((_ED * _PW, _C), jnp.float32),
        pltpu.VMEM((_PW, _C), jnp.float32),
        pltpu.VMEM((_PW, _C), jnp.float32),
        pltpu.SemaphoreType.DMA,
    ],
)
def _slice_step(yc, sims, bpad, out_hbm, idxs, rows, bww, outb, sem):
    wid = lax.axis_index("s") * _NC + lax.axis_index("c")
    pb = wid * _PCH
    pltpu.sync_copy(sims.at[pl.ds(_ED * pb, _ED * _PCH)], idxs)

    def win_body(w, carry):
        descs = [
            pltpu.async_copy(
                yc.at[idxs.at[pl.ds(w * (_ED * _PW) + j * _GB, _GB)]],
                rows.at[pl.ds(j * _GB, _GB)],
                sem,
            )
            for j in range(_ED * _PW // _GB)
        ]
        pltpu.sync_copy(bpad.at[pl.ds(pb + w * _PW, _PW)], bww)
        for dsc in descs:
            dsc.wait()

        def pt_body(i, c2):
            bv = bww[i, :]
            acc = bv[0] * rows[_ED * i, :]
            for j in range(1, _ED):
                acc = acc + bv[j] * rows[_ED * i + j, :]
            outb[i, :] = acc * _ALPHA
            return c2

        lax.fori_loop(0, _PW, pt_body, 0)
        pltpu.sync_copy(outb, out_hbm.at[pl.ds(pb + w * _PW, _PW)])
        return carry

    lax.fori_loop(0, _PCH // _PW, win_body, 0)


_Q = _M
_QCH = _Q // _NW
_QW = 512
_NSTEP = 20
_PAD1 = 16
_ROWS1 = _M + _PAD1


@functools.partial(
    pl.kernel,
    out_type=jax.ShapeDtypeStruct((_Q,), jnp.int32),
    mesh=_MESH,
    compiler_params=pltpu.CompilerParams(use_tc_tiling_on_sc=False),
    scratch_types=[
        pltpu.VMEM((_QW,), jnp.int32),
        pltpu.VMEM((_QW,), jnp.int32),
        pltpu.VMEM((_QW,), jnp.int32),
        pltpu.VMEM((_QW,), jnp.int32),
        pltpu.VMEM((_QW,), jnp.int32),
        pltpu.VMEM((_QW,), jnp.int32),
        pltpu.VMEM((_QW,), jnp.int32),
        pltpu.VMEM((16,), jnp.int32),
        pltpu.SemaphoreType.DMA,
    ],
)
def _lookup_sc(uk1, uk2, qk1, qk2, uarr, out, q1, q2, lo, hi, mid, mk1, mk2, ub, sem):
    wid = lax.axis_index("s") * _NC + lax.axis_index("c")
    pltpu.sync_copy(uarr, ub)
    u = ub[pl.ds(0, 16)][0]
    nblk = _QW // 16
    ndma = _QW // _GB

    def win_body(w, carry):
        o = (wid + _NW * w) * _QW
        pltpu.sync_copy(qk1.at[pl.ds(o, _QW)], q1)
        pltpu.sync_copy(qk2.at[pl.ds(o, _QW)], q2)

        def init_blk(i, c):
            s = pl.ds(i * 16, 16)
            lo[s] = jnp.zeros((16,), jnp.int32)
            hi[s] = jnp.full((16,), _M, jnp.int32)
            return c

        lax.fori_loop(0, nblk, init_blk, 0)

        def gather_mid():
            def fire(j, c):
                sl = pl.ds(j * _GB, _GB)
                pltpu.async_copy(uk1.at[mid.at[sl]], mk1.at[sl], sem)
                pltpu.async_copy(uk2.at[mid.at[sl]], mk2.at[sl], sem)
                return c

            lax.fori_loop(0, ndma, fire, 0)
            pltpu.make_async_copy(uk1.at[pl.ds(0, _QW)], mk1, sem).wait()
            pltpu.make_async_copy(uk2.at[pl.ds(0, _QW)], mk2, sem).wait()

        for _step in range(_NSTEP):
            def mid_blk(i, c):
                s = pl.ds(i * 16, 16)
                mid[s] = lax.shift_right_arithmetic(lo[s] + hi[s], 1)
                return c

            lax.fori_loop(0, nblk, mid_blk, 0)
            gather_mid()

            def upd_blk(i, c):
                s = pl.ds(i * 16, 16)
                m1 = mk1[s]
                m2 = mk2[s]
                less = (m1 < q1[s]) | ((m1 == q1[s]) & (m2 < q2[s]))
                mm = mid[s]
                lo[s] = jnp.where(less, mm + 1, lo[s])
                hi[s] = jnp.where(less, hi[s], mm)
                return c

            lax.fori_loop(0, nblk, upd_blk, 0)

        def clamp_blk(i, c):
            s = pl.ds(i * 16, 16)
            mid[s] = jnp.minimum(lo[s], _M - 1)
            return c

        lax.fori_loop(0, nblk, clamp_blk, 0)
        gather_mid()

        def res_blk(i, c):
            s = pl.ds(i * 16, 16)
            found = (lo[s] < _M) & (mk1[s] == q1[s]) & (mk2[s] == q2[s])
            mid[s] = jnp.where(found, lo[s], -1)
            return c

        lax.fori_loop(0, nblk, res_blk, 0)
        pltpu.sync_copy(mid, out.at[pl.ds(o, _QW)])
        return carry

    def win_guard(w, carry):
        @pl.when(wid * _QW + _NW * w * _QW < u)
        def _go():
            win_body(w, 0)
        return carry

    lax.fori_loop(0, _QCH // _QW, win_guard, 0)


@functools.partial(
    pl.kernel,
    out_type=jax.ShapeDtypeStruct((_ROWS1,), jnp.float32),
    mesh=_MESH,
    compiler_params=pltpu.CompilerParams(use_tc_tiling_on_sc=False),
    scratch_types=[
        pltpu.VMEM((_BRR,), jnp.int32),
        pltpu.VMEM((_BRR,), jnp.int32),
        pltpu.VMEM((_BRR,), jnp.float32),
        pltpu.VMEM((_BRR,), jnp.float32),
        pltpu.VMEM((_BRR,), jnp.float32),
        pltpu.VMEM((_BRR,), jnp.float32),
        pltpu.VMEM((16,), jnp.int32),
        pltpu.SemaphoreType.DMA,
    ],
)
def _blur1_step(w_in, nbp, nbm, uarr, w_out, ixp, ixm, gp, gm, old, ob, ub, sem):
    wid = lax.axis_index("s") * _NC + lax.axis_index("c")
    pltpu.sync_copy(uarr, ub)
    u = ub[pl.ds(0, 16)][0]

    @pl.when(wid == 0)
    def _zero_pad():
        ob[pl.ds(0, 16)] = jnp.zeros((16,), jnp.float32)
        pltpu.sync_copy(ob.at[pl.ds(0, 16)], w_out.at[pl.ds(0, _PAD1)])

    def blk_body(t, carry):
        row0 = (wid + _NW * t) * _BRR

        @pl.when(row0 < u)
        def _go():
            pltpu.sync_copy(nbp.at[pl.ds(row0, _BRR)], ixp)
            pltpu.sync_copy(nbm.at[pl.ds(row0, _BRR)], ixm)

            def fire(j, c):
                sl = pl.ds(j * _GB, _GB)
                pltpu.async_copy(w_in.at[ixp.at[sl]], gp.at[sl], sem)
                pltpu.async_copy(w_in.at[ixm.at[sl]], gm.at[sl], sem)
                return c

            lax.fori_loop(0, _BRR // _GB, fire, 0)
            pltpu.sync_copy(w_in.at[pl.ds(_PAD1 + row0, _BRR)], old)
            pltpu.make_async_copy(w_in.at[pl.ds(0, _BRR)], gp, sem).wait()
            pltpu.make_async_copy(w_in.at[pl.ds(0, _BRR)], gm, sem).wait()

            def blk(i, c):
                s = pl.ds(i * 16, 16)
                ob[s] = old[s] + 0.5 * (gp[s] + gm[s])
                return c

            lax.fori_loop(0, _BRR // 16, blk, 0)
            pltpu.sync_copy(ob, w_out.at[pl.ds(_PAD1 + row0, _BRR)])

        return carry

    lax.fori_loop(0, _CH // _BRR, blk_body, 0)


@functools.partial(
    pl.kernel,
    out_type=jax.ShapeDtypeStruct((_M,), jnp.float32),
    mesh=_MESH,
    compiler_params=pltpu.CompilerParams(use_tc_tiling_on_sc=False),
    scratch_types=[
        pltpu.VMEM((_CH,), jnp.int32),
        pltpu.VMEM((_CH,), jnp.float32),
        pltpu.SemaphoreType.DMA,
    ],
)
def _gather1(w_in, idx, out, ix, g, sem):
    wid = lax.axis_index("s") * _NC + lax.axis_index("c")
    base = wid * _CH
    pltpu.sync_copy(idx.at[pl.ds(base, _CH)], ix)

    def fire(j, c):
        sl = pl.ds(j * _GB, _GB)
        pltpu.async_copy(w_in.at[ix.at[sl]], g.at[sl], sem)
        return c

    lax.fori_loop(0, _CH // _GB, fire, 0)
    pltpu.make_async_copy(w_in.at[pl.ds(0, _CH)], g, sem).wait()
    pltpu.sync_copy(g, out.at[pl.ds(base, _CH)])


@functools.partial(
    pl.kernel,
    out_type=jax.ShapeDtypeStruct((_M,), jnp.int32),
    mesh=_MESH,
    compiler_params=pltpu.CompilerParams(use_tc_tiling_on_sc=False),
    scratch_types=[
        pltpu.VMEM((_CH,), jnp.int32),
        pltpu.VMEM((_CH,), jnp.int32),
        pltpu.SemaphoreType.DMA,
    ],
)
def _gather1i(tab, idx, out, ix, g, sem):
    wid = lax.axis_index("s") * _NC + lax.axis_index("c")
    base = wid * _CH
    pltpu.sync_copy(idx.at[pl.ds(base, _CH)], ix)

    def fire(j, c):
        sl = pl.ds(j * _GB, _GB)
        pltpu.async_copy(tab.at[ix.at[sl]], g.at[sl], sem)
        return c

    lax.fori_loop(0, _CH // _GB, fire, 0)
    pltpu.make_async_copy(tab.at[pl.ds(0, _CH)], g, sem).wait()
    pltpu.sync_copy(g, out.at[pl.ds(base, _CH)])


def _proj_matrix(d):
    a = np.triu(np.ones((d, d), dtype=np.float32), 1) - np.diag(np.arange(1, d + 1).astype(np.float32))
    a = np.concatenate([np.ones((1, d), dtype=np.float32), a], axis=0)
    b = np.diag((1.0 / np.sqrt((np.arange(1, d + 1) * np.arange(2, d + 2)).astype(np.float32))))
    return (a @ b).astype(np.float32)


def _canon_simplex(d):
    rows = [[i] * (d + 1 - i) + [-(d + 1 - i)] * i for i in range(d + 1)]
    return np.array(rows, dtype=np.int64).T


def _basis(d):
    ed = d + 1
    return (ed * np.eye(ed) - np.ones((ed, ed))).astype(np.int64)


def _coords(x, sigmas):
    n, d = x.shape
    ed = d + 1
    sc = x / jnp.asarray(sigmas).reshape(1, d)
    sc = sc / (math.sqrt(2.0 / 3.0) * ed)
    e = jnp.asarray(_proj_matrix(d))
    p = sc @ e.T
    l0 = jnp.floor(p / ed) * ed
    residual = p - l0
    indices = jnp.argsort(-residual, axis=1)
    ranks = jnp.argsort(indices, axis=1).astype(p.dtype)
    greedy = ranks + l0.sum(axis=1, keepdims=True) / ed
    l0 = jnp.where(greedy < 0, l0 + ed, jnp.where(greedy > d, l0 - ed, l0))
    ranks = jnp.where(greedy < 0, greedy + ed, jnp.where(greedy > d, greedy - ed, greedy))
    return p, l0, ranks


def _bary(x, sigmas):
    n, d = x.shape
    ed = d + 1
    p, l0, ranks = _coords(x, sigmas)
    residual = (p - l0) / ed
    order = jnp.argsort(-ranks, axis=1)
    g = jnp.take_along_axis(residual, order, axis=1)
    b = jnp.diff(g, axis=1)
    b = jnp.concatenate([1.0 - b.sum(axis=1, keepdims=True), b], axis=1)
    return b


def _pack(pts):
    s = pts.astype(jnp.int32) + 512
    ed = s.shape[-1]
    h = ed // 2
    k1 = s[..., 0]
    for j in range(1, h):
        k1 = k1 * 1024 + s[..., j]
    k2 = s[..., h]
    for j in range(h + 1, ed):
        k2 = k2 * 1024 + s[..., j]
    return k1, k2


def _lookup(uk1, uk2, qk1, qk2):
    mm = uk1.shape[0]
    lo = jnp.zeros(qk1.shape, dtype=jnp.int32)
    hi = jnp.full(qk1.shape, mm, dtype=jnp.int32)
    for _ in range(int(math.ceil(math.log2(mm))) + 1):
        mid = (lo + hi) // 2
        mk1 = uk1[mid]
        mk2 = uk2[mid]
        less = (mk1 < qk1) | ((mk1 == qk1) & (mk2 < qk2))
        lo = jnp.where(less, mid + 1, lo)
        hi = jnp.where(less, hi, mid)
    fk1 = uk1[jnp.minimum(lo, mm - 1)]
    fk2 = uk2[jnp.minimum(lo, mm - 1)]
    found = (lo < mm) & (fk1 == qk1) & (fk2 == qk2)
    return jnp.where(found, lo, -1)


def _fit(x, sigmas):
    n, d = x.shape
    ed = d + 1
    m = n * ed
    _, l0, ranks = _coords(x, sigmas)
    l0 = l0.astype(jnp.int32)
    ri = ranks.astype(jnp.int32)
    cs = jnp.asarray(_canon_simplex(d).astype(np.int32))
    pts = l0[:, None, :] + jnp.take(cs, ri, axis=1).transpose(1, 0, 2)
    pts_flat = pts.reshape(-1, ed)
    k1, k2 = _pack(pts_flat)
    perm = jnp.lexsort((k2, k1))
    zpad = jnp.zeros((_PAD1,), jnp.int32)
    sk1 = _gather1i(jnp.concatenate([zpad, k1]), perm + _PAD1)
    sk2 = _gather1i(jnp.concatenate([zpad, k2]), perm + _PAD1)
    new = jnp.concatenate([jnp.ones((1,), dtype=bool),
                           (sk1[1:] != sk1[:-1]) | (sk2[1:] != sk2[:-1])])
    ids_sorted = jnp.cumsum(new.astype(jnp.int32)) - 1
    uarr = jnp.full((16,), ids_sorted[-1] + 1, dtype=jnp.int32)
    inv = jnp.zeros((m,), dtype=jnp.int32).at[perm].set(ids_sorted)
    simplices = inv.reshape(n, ed)
    slot = jnp.where(new, ids_sorted, m)
    sentinel = jnp.iinfo(jnp.int32).max
    uk1 = jnp.full((m,), sentinel, dtype=jnp.int32).at[slot].set(sk1, mode='drop')
    uk2 = jnp.full((m,), sentinel, dtype=jnp.int32).at[slot].set(sk2, mode='drop')
    off = _basis(d).astype(np.int64)
    res = []
    for sgn in (1, -1):
        for k in range(ed):
            o = off[k]
            d1 = int(o[0]) * 1024 * 1024 + int(o[1]) * 1024 + int(o[2])
            d2 = int(o[3]) * 1024 * 1024 + int(o[4]) * 1024 + int(o[5])
            res.append(_lookup_sc(uk1, uk2,
                                  uk1 + jnp.int32(sgn * d1),
                                  uk2 + jnp.int32(sgn * d2), uarr))
    neighbors = jnp.stack(res, axis=1).reshape(m, 2, ed)
    return simplices, neighbors, uarr


def _filter_sc(yq, b, sims_flat, simsp1, nbis, bpad, uarr):
    yb = b[:, :, None] * yq[:, None, :]
    s = jnp.zeros((_M, _C), dtype=jnp.float32).at[sims_flat].add(yb.reshape(-1, _C))
    yc = jnp.concatenate([jnp.zeros((_PAD, _C), dtype=jnp.float32), s], axis=0)
    for dd in range(_ED):
        yc = _blur_step(yc, nbis[dd], uarr)
    return _slice_step(yc, simsp1, bpad)


def kernel(x, y):
    simplices, neighbors, uarr = _fit(x, _SIGMAS)
    b = _bary(x, _SIGMAS)
    sims_flat = simplices.reshape(-1)
    simsp1 = sims_flat + _PAD
    nbis = [(neighbors[:, :, dd] + _PAD).reshape(-1) for dd in range(_ED)]
    bpad = jnp.zeros((_N, _C), dtype=jnp.float32).at[:, :_ED].set(b)

    b_flat = b.reshape(-1)
    s1 = jnp.zeros((_M,), dtype=jnp.float32).at[sims_flat].add(b_flat)
    w = jnp.concatenate([jnp.zeros((_PAD1,), jnp.float32), s1])
    nbp = [neighbors[:, 0, dd] + _PAD1 for dd in range(_ED)]
    nbm = [neighbors[:, 1, dd] + _PAD1 for dd in range(_ED)]
    for dd in range(_ED):
        w = _blur1_step(w, nbp[dd], nbm[dd], uarr)
    g1 = _gather1(w, sims_flat + _PAD1)
    r1 = (b_flat * g1).reshape(_N, _ED).sum(axis=1) * _ALPHA
    norms = (1.0 / jnp.sqrt(r1 + 1e-20))[:, None]

    out = _filter_sc(y * norms, b, sims_flat, simsp1, nbis, bpad, uarr)
    return out * norms


# speedup vs baseline: 326.1942x; 1.0135x over previous
import functools
import math
import jax
import jax.numpy as jnp
import numpy as np
from jax import lax
from jax.experimental import pallas as pl
from jax.experimental.pallas import tpu as pltpu
from jax.experimental.pallas import tpu_sc as plsc

_SIGMAS = np.array([0.02, 0.02, 0.05, 0.05, 0.05], dtype=np.float32)

_N = 65536
_D = 5
_ED = 6
_C = 16
_M = _N * _ED
_PAD = 8
_ROWS = _M + _PAD
_NC, _NS = 2, 16
_NW = _NC * _NS
_CH = _M // _NW
_GB = 128
_BRR = 512
_VWIN = 1024
_PCH = _N // _NW
_PW = 256
_ALPHA = 1.0 / (1.0 + 2.0 ** (-_D))

_MESH = plsc.VectorSubcoreMesh(core_axis_name="c", subcore_axis_name="s")
_MESH1 = plsc.VectorSubcoreMesh(core_axis_name="c", subcore_axis_name="s",
                                num_cores=1)
_NW1 = _NS


@functools.partial(
    pl.kernel,
    out_type=jax.ShapeDtypeStruct((_ROWS, _C), jnp.float32),
    mesh=_MESH,
    compiler_params=pltpu.CompilerParams(use_tc_tiling_on_sc=False),
    scratch_types=[
        pltpu.VMEM((2 * _BRR,), jnp.int32),
        pltpu.VMEM((2 * _BRR, _C), jnp.float32),
        pltpu.VMEM((_BRR, _C), jnp.float32),
        pltpu.VMEM((_BRR, _C), jnp.float32),
        pltpu.VMEM((16,), jnp.int32),
        pltpu.SemaphoreType.DMA,
    ],
)
def _blur_step(yc_in, nbi, uarr, yc_out, idx_all, gath, old, outb, ub, sem):
    wid = lax.axis_index("s") * _NC + lax.axis_index("c")
    pltpu.sync_copy(uarr, ub)
    u = ub[pl.ds(0, 16)][0]

    @pl.when(wid == 0)
    def _zero_rows():
        for r in range(_PAD):
            outb[r, :] = jnp.zeros((_C,), jnp.float32)
        pltpu.sync_copy(outb.at[pl.ds(0, _PAD)], yc_out.at[pl.ds(0, _PAD)])

    def blk_body(t, carry):
        row0 = (wid + _NW * t) * _BRR

        @pl.when(row0 < u)
        def _go():
            pltpu.sync_copy(nbi.at[pl.ds(2 * row0, 2 * _BRR)], idx_all)
            descs = [
                pltpu.async_copy(
                    yc_in.at[idx_all.at[pl.ds(j * _GB, _GB)]],
                    gath.at[pl.ds(j * _GB, _GB)],
                    sem,
                )
                for j in range(2 * _BRR // _GB)
            ]
            pltpu.sync_copy(yc_in.at[pl.ds(_PAD + row0, _BRR)], old)
            for dsc in descs:
                dsc.wait()

            def row_body(i, c2):
                g0 = gath[2 * i, :]
                g1 = gath[2 * i + 1, :]
                outb[i, :] = old[i, :] + 0.5 * (g0 + g1)
                return c2

            lax.fori_loop(0, _BRR, row_body, 0)
            pltpu.sync_copy(outb.at[pl.ds(0, _BRR)], yc_out.at[pl.ds(_PAD + row0, _BRR)])

        return carry

    lax.fori_loop(0, _CH // _BRR, blk_body, 0)


@functools.partial(
    pl.kernel,
    out_type=jax.ShapeDtypeStruct((_N, _C), jnp.float32),
    mesh=_MESH,
    compiler_params=pltpu.CompilerParams(use_tc_tiling_on_sc=False),
    scratch_types=[
        pltpu.VMEM((_ED * _PCH,), jnp.int32),
        pltpu.VMEM((_ED * _PW, _C), jnp.float32),
        pltpu.VMEM((_PW, _C), jnp.float32),
        pltpu.VMEM((_PW, _C), jnp.float32),
        pltpu.SemaphoreType.DMA,
    ],
)
def _slice_step(yc, sims, bpad, out_hbm, idxs, rows, bww, outb, sem):
    wid = lax.axis_index("s") * _NC + lax.axis_index("c")
    pb = wid * _PCH
    pltpu.sync_copy(sims.at[pl.ds(_ED * pb, _ED * _PCH)], idxs)

    def win_body(w, carry):
        descs = [
            pltpu.async_copy(
                yc.at[idxs.at[pl.ds(w * (_ED * _PW) + j * _GB, _GB)]],
                rows.at[pl.ds(j * _GB, _GB)],
                sem,
            )
            for j in range(_ED * _PW // _GB)
        ]
        pltpu.sync_copy(bpad.at[pl.ds(pb + w * _PW, _PW)], bww)
        for dsc in descs:
            dsc.wait()

        def pt_body(i, c2):
            bv = bww[i, :]
            acc = bv[0] * rows[_ED * i, :]
            for j in range(1, _ED):
                acc = acc + bv[j] * rows[_ED * i + j, :]
            outb[i, :] = acc * _ALPHA
            return c2

        lax.fori_loop(0, _PW, pt_body, 0)
        pltpu.sync_copy(outb, out_hbm.at[pl.ds(pb + w * _PW, _PW)])
        return carry

    lax.fori_loop(0, _PCH // _PW, win_body, 0)


_Q = _M
_QCH = _Q // _NW
_QW = 512
_NSTEP = 20
_PAD1 = 16
_ROWS1 = _M + _PAD1


@functools.partial(
    pl.kernel,
    out_type=jax.ShapeDtypeStruct((_Q,), jnp.int32),
    mesh=_MESH,
    compiler_params=pltpu.CompilerParams(use_tc_tiling_on_sc=False),
    scratch_types=[
        pltpu.VMEM((_QW,), jnp.int32),
        pltpu.VMEM((_QW,), jnp.int32),
        pltpu.VMEM((_QW,), jnp.int32),
        pltpu.VMEM((_QW,), jnp.int32),
        pltpu.VMEM((_QW,), jnp.int32),
        pltpu.VMEM((_QW,), jnp.int32),
        pltpu.VMEM((_QW,), jnp.int32),
        pltpu.VMEM((16,), jnp.int32),
        pltpu.SemaphoreType.DMA,
    ],
)
def _lookup_sc(uk1, uk2, qk1, qk2, uarr, out, q1, q2, lo, hi, mid, mk1, mk2, ub, sem):
    wid = lax.axis_index("s") * _NC + lax.axis_index("c")
    pltpu.sync_copy(uarr, ub)
    u = ub[pl.ds(0, 16)][0]
    nblk = _QW // 16
    ndma = _QW // _GB

    def win_body(w, carry):
        o = (wid + _NW * w) * _QW
        pltpu.sync_copy(qk1.at[pl.ds(o, _QW)], q1)
        pltpu.sync_copy(qk2.at[pl.ds(o, _QW)], q2)

        def init_blk(i, c):
            s = pl.ds(i * 16, 16)
            lo[s] = jnp.zeros((16,), jnp.int32)
            hi[s] = jnp.full((16,), _M, jnp.int32)
            return c

        lax.fori_loop(0, nblk, init_blk, 0)

        def gather_mid():
            def fire(j, c):
                sl = pl.ds(j * _GB, _GB)
                pltpu.async_copy(uk1.at[mid.at[sl]], mk1.at[sl], sem)
                pltpu.async_copy(uk2.at[mid.at[sl]], mk2.at[sl], sem)
                return c

            lax.fori_loop(0, ndma, fire, 0)
            pltpu.make_async_copy(uk1.at[pl.ds(0, _QW)], mk1, sem).wait()
            pltpu.make_async_copy(uk2.at[pl.ds(0, _QW)], mk2, sem).wait()

        for _step in range(_NSTEP):
            def mid_blk(i, c):
                s = pl.ds(i * 16, 16)
                mid[s] = lax.shift_right_arithmetic(lo[s] + hi[s], 1)
                return c

            lax.fori_loop(0, nblk, mid_blk, 0)
            gather_mid()

            def upd_blk(i, c):
                s = pl.ds(i * 16, 16)
                m1 = mk1[s]
                m2 = mk2[s]
                less = (m1 < q1[s]) | ((m1 == q1[s]) & (m2 < q2[s]))
                mm = mid[s]
                lo[s] = jnp.where(less, mm + 1, lo[s])
                hi[s] = jnp.where(less, hi[s], mm)
                return c

            lax.fori_loop(0, nblk, upd_blk, 0)

        def clamp_blk(i, c):
            s = pl.ds(i * 16, 16)
            mid[s] = jnp.minimum(lo[s], _M - 1)
            return c

        lax.fori_loop(0, nblk, clamp_blk, 0)
        gather_mid()

        def res_blk(i, c):
            s = pl.ds(i * 16, 16)
            found = (lo[s] < _M) & (mk1[s] == q1[s]) & (mk2[s] == q2[s])
            mid[s] = jnp.where(found, lo[s], -1)
            return c

        lax.fori_loop(0, nblk, res_blk, 0)
        pltpu.sync_copy(mid, out.at[pl.ds(o, _QW)])
        return carry

    def win_guard(w, carry):
        @pl.when(wid * _QW + _NW * w * _QW < u)
        def _go():
            win_body(w, 0)
        return carry

    lax.fori_loop(0, _QCH // _QW, win_guard, 0)


@functools.partial(
    pl.kernel,
    out_type=jax.ShapeDtypeStruct((_ROWS1,), jnp.float32),
    mesh=_MESH,
    compiler_params=pltpu.CompilerParams(use_tc_tiling_on_sc=False),
    scratch_types=[
        pltpu.VMEM((_BRR,), jnp.int32),
        pltpu.VMEM((_BRR,), jnp.int32),
        pltpu.VMEM((_BRR,), jnp.float32),
        pltpu.VMEM((_BRR,), jnp.float32),
        pltpu.VMEM((_BRR,), jnp.float32),
        pltpu.VMEM((_BRR,), jnp.float32),
        pltpu.VMEM((16,), jnp.int32),
        pltpu.SemaphoreType.DMA,
    ],
)
def _blur1_step(w_in, nbp, nbm, uarr, w_out, ixp, ixm, gp, gm, old, ob, ub, sem):
    wid = lax.axis_index("s") * _NC + lax.axis_index("c")
    pltpu.sync_copy(uarr, ub)
    u = ub[pl.ds(0, 16)][0]

    @pl.when(wid == 0)
    def _zero_pad():
        ob[pl.ds(0, 16)] = jnp.zeros((16,), jnp.float32)
        pltpu.sync_copy(ob.at[pl.ds(0, 16)], w_out.at[pl.ds(0, _PAD1)])

    def blk_body(t, carry):
        row0 = (wid + _NW * t) * _BRR

        @pl.when(row0 < u)
        def _go():
            pltpu.sync_copy(nbp.at[pl.ds(row0, _BRR)], ixp)
            pltpu.sync_copy(nbm.at[pl.ds(row0, _BRR)], ixm)

            def fire(j, c):
                sl = pl.ds(j * _GB, _GB)
                pltpu.async_copy(w_in.at[ixp.at[sl]], gp.at[sl], sem)
                pltpu.async_copy(w_in.at[ixm.at[sl]], gm.at[sl], sem)
                return c

            lax.fori_loop(0, _BRR // _GB, fire, 0)
            pltpu.sync_copy(w_in.at[pl.ds(_PAD1 + row0, _BRR)], old)
            pltpu.make_async_copy(w_in.at[pl.ds(0, _BRR)], gp, sem).wait()
            pltpu.make_async_copy(w_in.at[pl.ds(0, _BRR)], gm, sem).wait()

            def blk(i, c):
                s = pl.ds(i * 16, 16)
                ob[s] = old[s] + 0.5 * (gp[s] + gm[s])
                return c

            lax.fori_loop(0, _BRR // 16, blk, 0)
            pltpu.sync_copy(ob, w_out.at[pl.ds(_PAD1 + row0, _BRR)])

        return carry

    lax.fori_loop(0, _CH // _BRR, blk_body, 0)


@functools.partial(
    pl.kernel,
    out_type=jax.ShapeDtypeStruct((_M,), jnp.float32),
    mesh=_MESH,
    compiler_params=pltpu.CompilerParams(use_tc_tiling_on_sc=False),
    scratch_types=[
        pltpu.VMEM((_CH,), jnp.int32),
        pltpu.VMEM((_CH,), jnp.float32),
        pltpu.SemaphoreType.DMA,
    ],
)
def _gather1(w_in, idx, out, ix, g, sem):
    wid = lax.axis_index("s") * _NC + lax.axis_index("c")
    base = wid * _CH
    pltpu.sync_copy(idx.at[pl.ds(base, _CH)], ix)

    def fire(j, c):
        sl = pl.ds(j * _GB, _GB)
        pltpu.async_copy(w_in.at[ix.at[sl]], g.at[sl], sem)
        return c

    lax.fori_loop(0, _CH // _GB, fire, 0)
    pltpu.make_async_copy(w_in.at[pl.ds(0, _CH)], g, sem).wait()
    pltpu.sync_copy(g, out.at[pl.ds(base, _CH)])


@functools.partial(
    pl.kernel,
    out_type=jax.ShapeDtypeStruct((_M,), jnp.int32),
    mesh=_MESH,
    compiler_params=pltpu.CompilerParams(use_tc_tiling_on_sc=False),
    scratch_types=[
        pltpu.VMEM((_CH,), jnp.int32),
        pltpu.VMEM((_CH,), jnp.int32),
        pltpu.SemaphoreType.DMA,
    ],
)
def _gather1i(tab, idx, out, ix, g, sem):
    wid = lax.axis_index("s") * _NC + lax.axis_index("c")
    base = wid * _CH
    pltpu.sync_copy(idx.at[pl.ds(base, _CH)], ix)

    def fire(j, c):
        sl = pl.ds(j * _GB, _GB)
        pltpu.async_copy(tab.at[ix.at[sl]], g.at[sl], sem)
        return c

    lax.fori_loop(0, _CH // _GB, fire, 0)
    pltpu.make_async_copy(tab.at[pl.ds(0, _CH)], g, sem).wait()
    pltpu.sync_copy(g, out.at[pl.ds(base, _CH)])


def _proj_matrix(d):
    a = np.triu(np.ones((d, d), dtype=np.float32), 1) - np.diag(np.arange(1, d + 1).astype(np.float32))
    a = np.concatenate([np.ones((1, d), dtype=np.float32), a], axis=0)
    b = np.diag((1.0 / np.sqrt((np.arange(1, d + 1) * np.arange(2, d + 2)).astype(np.float32))))
    return (a @ b).astype(np.float32)


def _canon_simplex(d):
    rows = [[i] * (d + 1 - i) + [-(d + 1 - i)] * i for i in range(d + 1)]
    return np.array(rows, dtype=np.int64).T


def _basis(d):
    ed = d + 1
    return (ed * np.eye(ed) - np.ones((ed, ed))).astype(np.int64)


def _coords(x, sigmas):
    n, d = x.shape
    ed = d + 1
    sc = x / jnp.asarray(sigmas).reshape(1, d)
    sc = sc / (math.sqrt(2.0 / 3.0) * ed)
    e = jnp.asarray(_proj_matrix(d))
    p = sc @ e.T
    l0 = jnp.floor(p / ed) * ed
    residual = p - l0
    indices = jnp.argsort(-residual, axis=1)
    ranks = jnp.argsort(indices, axis=1).astype(p.dtype)
    greedy = ranks + l0.sum(axis=1, keepdims=True) / ed
    l0 = jnp.where(greedy < 0, l0 + ed, jnp.where(greedy > d, l0 - ed, l0))
    ranks = jnp.where(greedy < 0, greedy + ed, jnp.where(greedy > d, greedy - ed, greedy))
    return p, l0, ranks


def _bary(x, sigmas):
    n, d = x.shape
    ed = d + 1
    p, l0, ranks = _coords(x, sigmas)
    residual = (p - l0) / ed
    order = jnp.argsort(-ranks, axis=1)
    g = jnp.take_along_axis(residual, order, axis=1)
    b = jnp.diff(g, axis=1)
    b = jnp.concatenate([1.0 - b.sum(axis=1, keepdims=True), b], axis=1)
    return b


def _pack(pts):
    s = pts.astype(jnp.int32) + 512
    ed = s.shape[-1]
    h = ed // 2
    k1 = s[..., 0]
    for j in range(1, h):
        k1 = k1 * 1024 + s[..., j]
    k2 = s[..., h]
    for j in range(h + 1, ed):
        k2 = k2 * 1024 + s[..., j]
    return k1, k2


def _lookup(uk1, uk2, qk1, qk2):
    mm = uk1.shape[0]
    lo = jnp.zeros(qk1.shape, dtype=jnp.int32)
    hi = jnp.full(qk1.shape, mm, dtype=jnp.int32)
    for _ in range(int(math.ceil(math.log2(mm))) + 1):
        mid = (lo + hi) // 2
        mk1 = uk1[mid]
        mk2 = uk2[mid]
        less = (mk1 < qk1) | ((mk1 == qk1) & (mk2 < qk2))
        lo = jnp.where(less, mid + 1, lo)
        hi = jnp.where(less, hi, mid)
    fk1 = uk1[jnp.minimum(lo, mm - 1)]
    fk2 = uk2[jnp.minimum(lo, mm - 1)]
    found = (lo < mm) & (fk1 == qk1) & (fk2 == qk2)
    return jnp.where(found, lo, -1)


def _fit(x, sigmas):
    n, d = x.shape
    ed = d + 1
    m = n * ed
    _, l0, ranks = _coords(x, sigmas)
    l0 = l0.astype(jnp.int32)
    ri = ranks.astype(jnp.int32)
    cs = jnp.asarray(_canon_simplex(d).astype(np.int32))
    pts = l0[:, None, :] + jnp.take(cs, ri, axis=1).transpose(1, 0, 2)
    pts_flat = pts.reshape(-1, ed)
    k1, k2 = _pack(pts_flat)
    perm = jnp.lexsort((k2, k1))
    zpad = jnp.zeros((_PAD1,), jnp.int32)
    sk1 = _gather1i(jnp.concatenate([zpad, k1]), perm + _PAD1)
    sk2 = _gather1i(jnp.concatenate([zpad, k2]), perm + _PAD1)
    new = jnp.concatenate([jnp.ones((1,), dtype=bool),
                           (sk1[1:] != sk1[:-1]) | (sk2[1:] != sk2[:-1])])
    ids_sorted = jnp.cumsum(new.astype(jnp.int32)) - 1
    uarr = jnp.full((16,), ids_sorted[-1] + 1, dtype=jnp.int32)
    inv = jnp.zeros((m,), dtype=jnp.int32).at[perm].set(ids_sorted)
    simplices = inv.reshape(n, ed)
    slot = jnp.where(new, ids_sorted, m)
    sentinel = jnp.iinfo(jnp.int32).max
    uk1 = jnp.full((m,), sentinel, dtype=jnp.int32).at[slot].set(sk1, mode='drop')
    uk2 = jnp.full((m,), sentinel, dtype=jnp.int32).at[slot].set(sk2, mode='drop')
    off = _basis(d).astype(np.int64)
    res = []
    for sgn in (1, -1):
        for k in range(ed):
            o = off[k]
            d1 = int(o[0]) * 1024 * 1024 + int(o[1]) * 1024 + int(o[2])
            d2 = int(o[3]) * 1024 * 1024 + int(o[4]) * 1024 + int(o[5])
            res.append(_lookup_sc(uk1, uk2,
                                  uk1 + jnp.int32(sgn * d1),
                                  uk2 + jnp.int32(sgn * d2), uarr))
    neighbors = jnp.stack(res, axis=1).reshape(m, 2, ed)
    return simplices, neighbors, uarr


@functools.partial(
    pl.kernel,
    out_type=(jax.ShapeDtypeStruct((_ROWS, _C), jnp.float32),
              jax.ShapeDtypeStruct((_ROWS, _C), jnp.float32)),
    mesh=_MESH1,
    compiler_params=pltpu.CompilerParams(use_tc_tiling_on_sc=False),
    scratch_types=[
        pltpu.VMEM((2 * _BRR,), jnp.int32),
        pltpu.VMEM((2 * _BRR, _C), jnp.float32),
        pltpu.VMEM((_BRR, _C), jnp.float32),
        pltpu.VMEM((_BRR, _C), jnp.float32),
        pltpu.VMEM((16,), jnp.int32),
        pltpu.SemaphoreType.DMA,
    ],
)
def _blur_chain(yc0, nb0, nb1, nb2, nb3, nb4, nb5, uarr,
                ya, yb_, idx_all, gath, old, outb, ub, sem):
    wid = lax.axis_index("s")
    pltpu.sync_copy(uarr, ub)
    u = ub[pl.ds(0, 16)][0]
    nbs = (nb0, nb1, nb2, nb3, nb4, nb5)
    chain = [(yc0, ya), (ya, yb_), (yb_, ya), (ya, yb_), (yb_, ya), (ya, yb_)]
    for ph in range(_ED):
        src_r, dst_r = chain[ph]
        nbi = nbs[ph]

        @pl.when(wid == 0)
        def _zero_rows():
            for r in range(_PAD):
                outb[r, :] = jnp.zeros((_C,), jnp.float32)
            pltpu.sync_copy(outb.at[pl.ds(0, _PAD)], dst_r.at[pl.ds(0, _PAD)])

        def blk_body(t, carry):
            row0 = (wid + _NW1 * t) * _BRR

            @pl.when(row0 < u)
            def _go():
                pltpu.sync_copy(nbi.at[pl.ds(2 * row0, 2 * _BRR)], idx_all)
                descs = [
                    pltpu.async_copy(
                        src_r.at[idx_all.at[pl.ds(j * _GB, _GB)]],
                        gath.at[pl.ds(j * _GB, _GB)],
                        sem,
                    )
                    for j in range(2 * _BRR // _GB)
                ]
                pltpu.sync_copy(src_r.at[pl.ds(_PAD + row0, _BRR)], old)
                for dsc in descs:
                    dsc.wait()

                def row_body(i, c2):
                    g0 = gath[2 * i, :]
                    g1 = gath[2 * i + 1, :]
                    outb[i, :] = old[i, :] + 0.5 * (g0 + g1)
                    return c2

                lax.fori_loop(0, _BRR, row_body, 0)
                pltpu.sync_copy(outb.at[pl.ds(0, _BRR)],
                                dst_r.at[pl.ds(_PAD + row0, _BRR)])

            return carry

        lax.fori_loop(0, _M // (_NW1 * _BRR), blk_body, 0)
        plsc.subcore_barrier()


def _filter_sc(yq, b, sims_flat, simsp1, nbis, bpad, uarr):
    yb = b[:, :, None] * yq[:, None, :]
    s = jnp.zeros((_M, _C), dtype=jnp.float32).at[sims_flat].add(yb.reshape(-1, _C))
    yc = jnp.concatenate([jnp.zeros((_PAD, _C), dtype=jnp.float32), s], axis=0)
    ya, yb2 = _blur_chain(yc, nbis[0], nbis[1], nbis[2], nbis[3], nbis[4],
                          nbis[5], uarr)
    return _slice_step(yb2, simsp1, bpad)


@functools.partial(
    pl.kernel,
    out_type=(jax.ShapeDtypeStruct((_ROWS1,), jnp.float32),
              jax.ShapeDtypeStruct((_ROWS1,), jnp.float32)),
    mesh=_MESH1,
    compiler_params=pltpu.CompilerParams(use_tc_tiling_on_sc=False),
    scratch_types=[
        pltpu.VMEM((_BRR,), jnp.int32),
        pltpu.VMEM((_BRR,), jnp.int32),
        pltpu.VMEM((_BRR,), jnp.float32),
        pltpu.VMEM((_BRR,), jnp.float32),
        pltpu.VMEM((_BRR,), jnp.float32),
        pltpu.VMEM((_BRR,), jnp.float32),
        pltpu.VMEM((16,), jnp.int32),
        pltpu.SemaphoreType.DMA,
    ],
)
def _blur1_chain(w0, p0, p1, p2, p3, p4, p5, m0, m1, m2, m3, m4, m5, uarr,
                 wa, wb, ixp, ixm, gp, gm, old, ob, ub, sem):
    wid = lax.axis_index("s")
    pltpu.sync_copy(uarr, ub)
    u = ub[pl.ds(0, 16)][0]
    ps = (p0, p1, p2, p3, p4, p5)
    ms = (m0, m1, m2, m3, m4, m5)
    chain = [(w0, wa), (wa, wb), (wb, wa), (wa, wb), (wb, wa), (wa, wb)]
    for ph in range(_ED):
        src_r, dst_r = chain[ph]
        nbp_, nbm_ = ps[ph], ms[ph]

        @pl.when(wid == 0)
        def _zero_pad():
            ob[pl.ds(0, 16)] = jnp.zeros((16,), jnp.float32)
            pltpu.sync_copy(ob.at[pl.ds(0, 16)], dst_r.at[pl.ds(0, _PAD1)])

        def blk_body(t, carry):
            row0 = (wid + _NW1 * t) * _BRR

            @pl.when(row0 < u)
            def _go():
                pltpu.sync_copy(nbp_.at[pl.ds(row0, _BRR)], ixp)
                pltpu.sync_copy(nbm_.at[pl.ds(row0, _BRR)], ixm)

                def fire(j, c):
                    sl = pl.ds(j * _GB, _GB)
                    pltpu.async_copy(src_r.at[ixp.at[sl]], gp.at[sl], sem)
                    pltpu.async_copy(src_r.at[ixm.at[sl]], gm.at[sl], sem)
                    return c

                lax.fori_loop(0, _BRR // _GB, fire, 0)
                pltpu.sync_copy(src_r.at[pl.ds(_PAD1 + row0, _BRR)], old)
                pltpu.make_async_copy(src_r.at[pl.ds(0, _BRR)], gp, sem).wait()
                pltpu.make_async_copy(src_r.at[pl.ds(0, _BRR)], gm, sem).wait()

                def blk(i, c):
                    s = pl.ds(i * 16, 16)
                    ob[s] = old[s] + 0.5 * (gp[s] + gm[s])
                    return c

                lax.fori_loop(0, _BRR // 16, blk, 0)
                pltpu.sync_copy(ob, dst_r.at[pl.ds(_PAD1 + row0, _BRR)])

            return carry

        lax.fori_loop(0, _M // (_NW1 * _BRR), blk_body, 0)
        plsc.subcore_barrier()


def kernel(x, y):
    simplices, neighbors, uarr = _fit(x, _SIGMAS)
    b = _bary(x, _SIGMAS)
    sims_flat = simplices.reshape(-1)
    simsp1 = sims_flat + _PAD
    nbis = [(neighbors[:, :, dd] + _PAD).reshape(-1) for dd in range(_ED)]
    bpad = jnp.zeros((_N, _C), dtype=jnp.float32).at[:, :_ED].set(b)

    b_flat = b.reshape(-1)
    s1 = jnp.zeros((_M,), dtype=jnp.float32).at[sims_flat].add(b_flat)
    w = jnp.concatenate([jnp.zeros((_PAD1,), jnp.float32), s1])
    nbp = [neighbors[:, 0, dd] + _PAD1 for dd in range(_ED)]
    nbm = [neighbors[:, 1, dd] + _PAD1 for dd in range(_ED)]
    wa, wb = _blur1_chain(w, nbp[0], nbp[1], nbp[2], nbp[3], nbp[4], nbp[5],
                          nbm[0], nbm[1], nbm[2], nbm[3], nbm[4], nbm[5], uarr)
    g1 = _gather1(wb, sims_flat + _PAD1)
    r1 = (b_flat * g1).reshape(_N, _ED).sum(axis=1) * _ALPHA
    norms = (1.0 / jnp.sqrt(r1 + 1e-20))[:, None]

    out = _filter_sc(y * norms, b, sims_flat, simsp1, nbis, bpad, uarr)
    return out * norms


# speedup vs baseline: 377.6111x; 1.1576x over previous
import functools
import math
import jax
import jax.numpy as jnp
import numpy as np
from jax import lax
from jax.experimental import pallas as pl
from jax.experimental.pallas import tpu as pltpu
from jax.experimental.pallas import tpu_sc as plsc

_SIGMAS = np.array([0.02, 0.02, 0.05, 0.05, 0.05], dtype=np.float32)

_N = 65536
_D = 5
_ED = 6
_C = 16
_M = _N * _ED
_PAD = 8
_ROWS = _M + _PAD
_NC, _NS = 2, 16
_NW = _NC * _NS
_CH = _M // _NW
_GB = 128
_BRR = 512
_VWIN = 1024
_PCH = _N // _NW
_PW = 256
_ALPHA = 1.0 / (1.0 + 2.0 ** (-_D))

_MESH = plsc.VectorSubcoreMesh(core_axis_name="c", subcore_axis_name="s")
_MESH1 = plsc.VectorSubcoreMesh(core_axis_name="c", subcore_axis_name="s",
                                num_cores=1)
_NW1 = _NS


@functools.partial(
    pl.kernel,
    out_type=jax.ShapeDtypeStruct((_ROWS, _C), jnp.float32),
    mesh=_MESH,
    compiler_params=pltpu.CompilerParams(use_tc_tiling_on_sc=False),
    scratch_types=[
        pltpu.VMEM((2 * _BRR,), jnp.int32),
        pltpu.VMEM((2 * _BRR, _C), jnp.float32),
        pltpu.VMEM((_BRR, _C), jnp.float32),
        pltpu.VMEM((_BRR, _C), jnp.float32),
        pltpu.VMEM((16,), jnp.int32),
        pltpu.SemaphoreType.DMA,
    ],
)
def _blur_step(yc_in, nbi, uarr, yc_out, idx_all, gath, old, outb, ub, sem):
    wid = lax.axis_index("s") * _NC + lax.axis_index("c")
    pltpu.sync_copy(uarr, ub)
    u = ub[pl.ds(0, 16)][0]

    @pl.when(wid == 0)
    def _zero_rows():
        for r in range(_PAD):
            outb[r, :] = jnp.zeros((_C,), jnp.float32)
        pltpu.sync_copy(outb.at[pl.ds(0, _PAD)], yc_out.at[pl.ds(0, _PAD)])

    def blk_body(t, carry):
        row0 = (wid + _NW * t) * _BRR

        @pl.when(row0 < u)
        def _go():
            pltpu.sync_copy(nbi.at[pl.ds(2 * row0, 2 * _BRR)], idx_all)
            descs = [
                pltpu.async_copy(
                    yc_in.at[idx_all.at[pl.ds(j * _GB, _GB)]],
                    gath.at[pl.ds(j * _GB, _GB)],
                    sem,
                )
                for j in range(2 * _BRR // _GB)
            ]
            pltpu.sync_copy(yc_in.at[pl.ds(_PAD + row0, _BRR)], old)
            for dsc in descs:
                dsc.wait()

            def row_body(i, c2):
                g0 = gath[2 * i, :]
                g1 = gath[2 * i + 1, :]
                outb[i, :] = old[i, :] + 0.5 * (g0 + g1)
                return c2

            lax.fori_loop(0, _BRR, row_body, 0)
            pltpu.sync_copy(outb.at[pl.ds(0, _BRR)], yc_out.at[pl.ds(_PAD + row0, _BRR)])

        return carry

    lax.fori_loop(0, _CH // _BRR, blk_body, 0)


@functools.partial(
    pl.kernel,
    out_type=jax.ShapeDtypeStruct((_N, _C), jnp.float32),
    mesh=_MESH,
    compiler_params=pltpu.CompilerParams(use_tc_tiling_on_sc=False),
    scratch_types=[
        pltpu.VMEM((_ED * _PCH,), jnp.int32),
        pltpu.VMEM((_ED * _PW, _C), jnp.float32),
        pltpu.VMEM((_PW, _C), jnp.float32),
        pltpu.VMEM((_PW, _C), jnp.float32),
        pltpu.SemaphoreType.DMA,
    ],
)
def _slice_step(yc, sims, bpad, out_hbm, idxs, rows, bww, outb, sem):
    wid = lax.axis_index("s") * _NC + lax.axis_index("c")
    pb = wid * _PCH
    pltpu.sync_copy(sims.at[pl.ds(_ED * pb, _ED * _PCH)], idxs)

    def win_body(w, carry):
        descs = [
            pltpu.async_copy(
                yc.at[idxs.at[pl.ds(w * (_ED * _PW) + j * _GB, _GB)]],
                rows.at[pl.ds(j * _GB, _GB)],
                sem,
            )
            for j in range(_ED * _PW // _GB)
        ]
        pltpu.sync_copy(bpad.at[pl.ds(pb + w * _PW, _PW)], bww)
        for dsc in descs:
            dsc.wait()

        def pt_body(i, c2):
            bv = bww[i, :]
            acc = bv[0] * rows[_ED * i, :]
            for j in range(1, _ED):
                acc = acc + bv[j] * rows[_ED * i + j, :]
            outb[i, :] = acc * _ALPHA
            return c2

        lax.fori_loop(0, _PW, pt_body, 0)
        pltpu.sync_copy(outb, out_hbm.at[pl.ds(pb + w * _PW, _PW)])
        return carry

    lax.fori_loop(0, _PCH // _PW, win_body, 0)


_Q = _M
_QCH = _Q // _NW
_QW = 512
_NSTEP = 20
_PAD1 = 16
_ROWS1 = _M + _PAD1


def _make_lookup(deltas):
    nco = len(deltas)

    @functools.partial(
        pl.kernel,
        out_type=jax.ShapeDtypeStruct((nco * _M,), jnp.int32),
        mesh=_MESH,
        compiler_params=pltpu.CompilerParams(use_tc_tiling_on_sc=False),
        scratch_types=[
            pltpu.VMEM((_QW,), jnp.int32),
            pltpu.VMEM((_QW,), jnp.int32),
            pltpu.VMEM((_QW,), jnp.int32),
            pltpu.VMEM((_QW,), jnp.int32),
            pltpu.VMEM((_QW,), jnp.int32),
            pltpu.VMEM((_QW,), jnp.int32),
            pltpu.VMEM((_QW,), jnp.int32),
            pltpu.VMEM((16,), jnp.int32),
            pltpu.SemaphoreType.DMA,
        ],
    )
    def _lk(uk1, uk2, uarr, out, q1, q2, lo, hi, mid, mk1, mk2, ub, sem):
        wid = lax.axis_index("s") * _NC + lax.axis_index("c")
        pltpu.sync_copy(uarr, ub)
        u = ub[pl.ds(0, 16)][0]
        nblk = _QW // 16
        ndma = _QW // _GB

        def win_body(w):
            vb = (wid + _NW * w) * _QW
            pltpu.sync_copy(uk1.at[pl.ds(vb, _QW)], q1)
            pltpu.sync_copy(uk2.at[pl.ds(vb, _QW)], q2)

            def gather_mid():
                def fire(j, c):
                    sl = pl.ds(j * _GB, _GB)
                    pltpu.async_copy(uk1.at[mid.at[sl]], mk1.at[sl], sem)
                    pltpu.async_copy(uk2.at[mid.at[sl]], mk2.at[sl], sem)
                    return c

                lax.fori_loop(0, ndma, fire, 0)
                pltpu.make_async_copy(uk1.at[pl.ds(0, _QW)], mk1, sem).wait()
                pltpu.make_async_copy(uk2.at[pl.ds(0, _QW)], mk2, sem).wait()

            for d1, d2 in deltas:
                def init_blk(i, c):
                    s = pl.ds(i * 16, 16)
                    lo[s] = jnp.zeros((16,), jnp.int32)
                    hi[s] = jnp.full((16,), _M, jnp.int32)
                    return c

                lax.fori_loop(0, nblk, init_blk, 0)

                for _step in range(_NSTEP):
                    def mid_blk(i, c):
                        s = pl.ds(i * 16, 16)
                        mid[s] = lax.shift_right_arithmetic(lo[s] + hi[s], 1)
                        return c

                    lax.fori_loop(0, nblk, mid_blk, 0)
                    gather_mid()

                    def upd_blk(i, c):
                        s = pl.ds(i * 16, 16)
                        m1 = mk1[s]
                        m2 = mk2[s]
                        q1c = q1[s] + d1
                        q2c = q2[s] + d2
                        less = (m1 < q1c) | ((m1 == q1c) & (m2 < q2c))
                        mm = mid[s]
                        lo[s] = jnp.where(less, mm + 1, lo[s])
                        hi[s] = jnp.where(less, hi[s], mm)
                        return c

                    lax.fori_loop(0, nblk, upd_blk, 0)

                def clamp_blk(i, c):
                    s = pl.ds(i * 16, 16)
                    mid[s] = jnp.minimum(lo[s], _M - 1)
                    return c

                lax.fori_loop(0, nblk, clamp_blk, 0)
                gather_mid()

                def res_blk(i, c):
                    s = pl.ds(i * 16, 16)
                    found = (lo[s] < _M) & (mk1[s] == q1[s] + d1) & (mk2[s] == q2[s] + d2)
                    mid[s] = jnp.where(found, lo[s], -1)
                    return c

                lax.fori_loop(0, nblk, res_blk, 0)
                ci = deltas.index((d1, d2))
                pltpu.sync_copy(mid, out.at[pl.ds(ci * _M + vb, _QW)])

        def win_guard(w, carry):
            @pl.when((wid + _NW * w) * _QW < u)
            def _go():
                win_body(w)

            return carry

        lax.fori_loop(0, _QCH // _QW, win_guard, 0)

    return _lk


@functools.partial(
    pl.kernel,
    out_type=jax.ShapeDtypeStruct((_ROWS1,), jnp.float32),
    mesh=_MESH,
    compiler_params=pltpu.CompilerParams(use_tc_tiling_on_sc=False),
    scratch_types=[
        pltpu.VMEM((_BRR,), jnp.int32),
        pltpu.VMEM((_BRR,), jnp.int32),
        pltpu.VMEM((_BRR,), jnp.float32),
        pltpu.VMEM((_BRR,), jnp.float32),
        pltpu.VMEM((_BRR,), jnp.float32),
        pltpu.VMEM((_BRR,), jnp.float32),
        pltpu.VMEM((16,), jnp.int32),
        pltpu.SemaphoreType.DMA,
    ],
)
def _blur1_step(w_in, nbp, nbm, uarr, w_out, ixp, ixm, gp, gm, old, ob, ub, sem):
    wid = lax.axis_index("s") * _NC + lax.axis_index("c")
    pltpu.sync_copy(uarr, ub)
    u = ub[pl.ds(0, 16)][0]

    @pl.when(wid == 0)
    def _zero_pad():
        ob[pl.ds(0, 16)] = jnp.zeros((16,), jnp.float32)
        pltpu.sync_copy(ob.at[pl.ds(0, 16)], w_out.at[pl.ds(0, _PAD1)])

    def blk_body(t, carry):
        row0 = (wid + _NW * t) * _BRR

        @pl.when(row0 < u)
        def _go():
            pltpu.sync_copy(nbp.at[pl.ds(row0, _BRR)], ixp)
            pltpu.sync_copy(nbm.at[pl.ds(row0, _BRR)], ixm)

            def fire(j, c):
                sl = pl.ds(j * _GB, _GB)
                pltpu.async_copy(w_in.at[ixp.at[sl]], gp.at[sl], sem)
                pltpu.async_copy(w_in.at[ixm.at[sl]], gm.at[sl], sem)
                return c

            lax.fori_loop(0, _BRR // _GB, fire, 0)
            pltpu.sync_copy(w_in.at[pl.ds(_PAD1 + row0, _BRR)], old)
            pltpu.make_async_copy(w_in.at[pl.ds(0, _BRR)], gp, sem).wait()
            pltpu.make_async_copy(w_in.at[pl.ds(0, _BRR)], gm, sem).wait()

            def blk(i, c):
                s = pl.ds(i * 16, 16)
                ob[s] = old[s] + 0.5 * (gp[s] + gm[s])
                return c

            lax.fori_loop(0, _BRR // 16, blk, 0)
            pltpu.sync_copy(ob, w_out.at[pl.ds(_PAD1 + row0, _BRR)])

        return carry

    lax.fori_loop(0, _CH // _BRR, blk_body, 0)


@functools.partial(
    pl.kernel,
    out_type=jax.ShapeDtypeStruct((_M,), jnp.float32),
    mesh=_MESH,
    compiler_params=pltpu.CompilerParams(use_tc_tiling_on_sc=False),
    scratch_types=[
        pltpu.VMEM((_CH,), jnp.int32),
        pltpu.VMEM((_CH,), jnp.float32),
        pltpu.SemaphoreType.DMA,
    ],
)
def _gather1(w_in, idx, out, ix, g, sem):
    wid = lax.axis_index("s") * _NC + lax.axis_index("c")
    base = wid * _CH
    pltpu.sync_copy(idx.at[pl.ds(base, _CH)], ix)

    def fire(j, c):
        sl = pl.ds(j * _GB, _GB)
        pltpu.async_copy(w_in.at[ix.at[sl]], g.at[sl], sem)
        return c

    lax.fori_loop(0, _CH // _GB, fire, 0)
    pltpu.make_async_copy(w_in.at[pl.ds(0, _CH)], g, sem).wait()
    pltpu.sync_copy(g, out.at[pl.ds(base, _CH)])


@functools.partial(
    pl.kernel,
    out_type=jax.ShapeDtypeStruct((_M,), jnp.int32),
    mesh=_MESH,
    compiler_params=pltpu.CompilerParams(use_tc_tiling_on_sc=False),
    scratch_types=[
        pltpu.VMEM((_CH,), jnp.int32),
        pltpu.VMEM((_CH,), jnp.int32),
        pltpu.SemaphoreType.DMA,
    ],
)
def _gather1i(tab, idx, out, ix, g, sem):
    wid = lax.axis_index("s") * _NC + lax.axis_index("c")
    base = wid * _CH
    pltpu.sync_copy(idx.at[pl.ds(base, _CH)], ix)

    def fire(j, c):
        sl = pl.ds(j * _GB, _GB)
        pltpu.async_copy(tab.at[ix.at[sl]], g.at[sl], sem)
        return c

    lax.fori_loop(0, _CH // _GB, fire, 0)
    pltpu.make_async_copy(tab.at[pl.ds(0, _CH)], g, sem).wait()
    pltpu.sync_copy(g, out.at[pl.ds(base, _CH)])


def _proj_matrix(d):
    a = np.triu(np.ones((d, d), dtype=np.float32), 1) - np.diag(np.arange(1, d + 1).astype(np.float32))
    a = np.concatenate([np.ones((1, d), dtype=np.float32), a], axis=0)
    b = np.diag((1.0 / np.sqrt((np.arange(1, d + 1) * np.arange(2, d + 2)).astype(np.float32))))
    return (a @ b).astype(np.float32)


def _canon_simplex(d):
    rows = [[i] * (d + 1 - i) + [-(d + 1 - i)] * i for i in range(d + 1)]
    return np.array(rows, dtype=np.int64).T


def _basis(d):
    ed = d + 1
    return (ed * np.eye(ed) - np.ones((ed, ed))).astype(np.int64)


def _all_deltas():
    off = _basis(_D).astype(np.int64)
    ds_ = []
    for sgn in (1, -1):
        for k in range(_ED):
            o = off[k]
            d1 = int(o[0]) * 1024 * 1024 + int(o[1]) * 1024 + int(o[2])
            d2 = int(o[3]) * 1024 * 1024 + int(o[4]) * 1024 + int(o[5])
            ds_.append((sgn * d1, sgn * d2))
    return ds_


_DELTAS12 = _all_deltas()
_LOOKUP4S = [_make_lookup(_DELTAS12[4 * g:4 * g + 4]) for g in range(3)]


def _coords(x, sigmas):
    n, d = x.shape
    ed = d + 1
    sc = x / jnp.asarray(sigmas).reshape(1, d)
    sc = sc / (math.sqrt(2.0 / 3.0) * ed)
    e = jnp.asarray(_proj_matrix(d))
    p = sc @ e.T
    l0 = jnp.floor(p / ed) * ed
    residual = p - l0
    indices = jnp.argsort(-residual, axis=1)
    ranks = jnp.argsort(indices, axis=1).astype(p.dtype)
    greedy = ranks + l0.sum(axis=1, keepdims=True) / ed
    l0 = jnp.where(greedy < 0, l0 + ed, jnp.where(greedy > d, l0 - ed, l0))
    ranks = jnp.where(greedy < 0, greedy + ed, jnp.where(greedy > d, greedy - ed, greedy))
    return p, l0, ranks


def _bary(x, sigmas):
    n, d = x.shape
    ed = d + 1
    p, l0, ranks = _coords(x, sigmas)
    residual = (p - l0) / ed
    order = jnp.argsort(-ranks, axis=1)
    g = jnp.take_along_axis(residual, order, axis=1)
    b = jnp.diff(g, axis=1)
    b = jnp.concatenate([1.0 - b.sum(axis=1, keepdims=True), b], axis=1)
    return b


def _pack(pts):
    s = pts.astype(jnp.int32) + 512
    ed = s.shape[-1]
    h = ed // 2
    k1 = s[..., 0]
    for j in range(1, h):
        k1 = k1 * 1024 + s[..., j]
    k2 = s[..., h]
    for j in range(h + 1, ed):
        k2 = k2 * 1024 + s[..., j]
    return k1, k2


def _lookup(uk1, uk2, qk1, qk2):
    mm = uk1.shape[0]
    lo = jnp.zeros(qk1.shape, dtype=jnp.int32)
    hi = jnp.full(qk1.shape, mm, dtype=jnp.int32)
    for _ in range(int(math.ceil(math.log2(mm))) + 1):
        mid = (lo + hi) // 2
        mk1 = uk1[mid]
        mk2 = uk2[mid]
        less = (mk1 < qk1) | ((mk1 == qk1) & (mk2 < qk2))
        lo = jnp.where(less, mid + 1, lo)
        hi = jnp.where(less, hi, mid)
    fk1 = uk1[jnp.minimum(lo, mm - 1)]
    fk2 = uk2[jnp.minimum(lo, mm - 1)]
    found = (lo < mm) & (fk1 == qk1) & (fk2 == qk2)
    return jnp.where(found, lo, -1)


def _fit(x, sigmas):
    n, d = x.shape
    ed = d + 1
    m = n * ed
    _, l0, ranks = _coords(x, sigmas)
    l0 = l0.astype(jnp.int32)
    ri = ranks.astype(jnp.int32)
    cs = jnp.asarray(_canon_simplex(d).astype(np.int32))
    pts = l0[:, None, :] + jnp.take(cs, ri, axis=1).transpose(1, 0, 2)
    pts_flat = pts.reshape(-1, ed)
    k1, k2 = _pack(pts_flat)
    perm = jnp.lexsort((k2, k1))
    zpad = jnp.zeros((_PAD1,), jnp.int32)
    sk1 = _gather1i(jnp.concatenate([zpad, k1]), perm + _PAD1)
    sk2 = _gather1i(jnp.concatenate([zpad, k2]), perm + _PAD1)
    new = jnp.concatenate([jnp.ones((1,), dtype=bool),
                           (sk1[1:] != sk1[:-1]) | (sk2[1:] != sk2[:-1])])
    ids_sorted = jnp.cumsum(new.astype(jnp.int32)) - 1
    uarr = jnp.full((16,), ids_sorted[-1] + 1, dtype=jnp.int32)
    inv = jnp.zeros((m,), dtype=jnp.int32).at[perm].set(ids_sorted)
    simplices = inv.reshape(n, ed)
    slot = jnp.where(new, ids_sorted, m)
    sentinel = jnp.iinfo(jnp.int32).max
    uk1 = jnp.full((m,), sentinel, dtype=jnp.int32).at[slot].set(sk1, mode='drop')
    uk2 = jnp.full((m,), sentinel, dtype=jnp.int32).at[slot].set(sk2, mode='drop')
    off = _basis(d).astype(np.int64)
    deltas = []
    for sgn in (1, -1):
        for k in range(ed):
            o = off[k]
            d1 = int(o[0]) * 1024 * 1024 + int(o[1]) * 1024 + int(o[2])
            d2 = int(o[3]) * 1024 * 1024 + int(o[4]) * 1024 + int(o[5])
            deltas.append((sgn * d1, sgn * d2))
    parts = [_LOOKUP4S[g](uk1, uk2, uarr).reshape(4, m) for g in range(3)]
    nb12 = jnp.concatenate(parts, axis=0)
    neighbors = nb12.T.reshape(m, 2, ed)
    return simplices, neighbors, uarr


@functools.partial(
    pl.kernel,
    out_type=(jax.ShapeDtypeStruct((_ROWS, _C), jnp.float32),
              jax.ShapeDtypeStruct((_ROWS, _C), jnp.float32)),
    mesh=_MESH1,
    compiler_params=pltpu.CompilerParams(use_tc_tiling_on_sc=False),
    scratch_types=[
        pltpu.VMEM((2 * _BRR,), jnp.int32),
        pltpu.VMEM((2 * _BRR, _C), jnp.float32),
        pltpu.VMEM((_BRR, _C), jnp.float32),
        pltpu.VMEM((_BRR, _C), jnp.float32),
        pltpu.VMEM((16,), jnp.int32),
        pltpu.SemaphoreType.DMA,
    ],
)
def _blur_chain(yc0, nb0, nb1, nb2, nb3, nb4, nb5, uarr,
                ya, yb_, idx_all, gath, old, outb, ub, sem):
    wid = lax.axis_index("s")
    pltpu.sync_copy(uarr, ub)
    u = ub[pl.ds(0, 16)][0]
    nbs = (nb0, nb1, nb2, nb3, nb4, nb5)
    chain = [(yc0, ya), (ya, yb_), (yb_, ya), (ya, yb_), (yb_, ya), (ya, yb_)]
    for ph in range(_ED):
        src_r, dst_r = chain[ph]
        nbi = nbs[ph]

        @pl.when(wid == 0)
        def _zero_rows():
            for r in range(_PAD):
                outb[r, :] = jnp.zeros((_C,), jnp.float32)
            pltpu.sync_copy(outb.at[pl.ds(0, _PAD)], dst_r.at[pl.ds(0, _PAD)])

        def blk_body(t, carry):
            row0 = (wid + _NW1 * t) * _BRR

            @pl.when(row0 < u)
            def _go():
                pltpu.sync_copy(nbi.at[pl.ds(2 * row0, 2 * _BRR)], idx_all)
                descs = [
                    pltpu.async_copy(
                        src_r.at[idx_all.at[pl.ds(j * _GB, _GB)]],
                        gath.at[pl.ds(j * _GB, _GB)],
                        sem,
                    )
                    for j in range(2 * _BRR // _GB)
                ]
                pltpu.sync_copy(src_r.at[pl.ds(_PAD + row0, _BRR)], old)
                for dsc in descs:
                    dsc.wait()

                def row_body(i, c2):
                    g0 = gath[2 * i, :]
                    g1 = gath[2 * i + 1, :]
                    outb[i, :] = old[i, :] + 0.5 * (g0 + g1)
                    return c2

                lax.fori_loop(0, _BRR, row_body, 0)
                pltpu.sync_copy(outb.at[pl.ds(0, _BRR)],
                                dst_r.at[pl.ds(_PAD + row0, _BRR)])

            return carry

        lax.fori_loop(0, _M // (_NW1 * _BRR), blk_body, 0)
        plsc.subcore_barrier()


def _filter_sc(yq, b, sims_flat, simsp1, nbis, bpad, uarr):
    yb = b[:, :, None] * yq[:, None, :]
    s = jnp.zeros((_M, _C), dtype=jnp.float32).at[sims_flat].add(yb.reshape(-1, _C))
    yc = jnp.concatenate([jnp.zeros((_PAD, _C), dtype=jnp.float32), s], axis=0)
    ya, yb2 = _blur_chain(yc, nbis[0], nbis[1], nbis[2], nbis[3], nbis[4],
                          nbis[5], uarr)
    return _slice_step(yb2, simsp1, bpad)


@functools.partial(
    pl.kernel,
    out_type=(jax.ShapeDtypeStruct((_ROWS1,), jnp.float32),
              jax.ShapeDtypeStruct((_ROWS1,), jnp.float32)),
    mesh=_MESH1,
    compiler_params=pltpu.CompilerParams(use_tc_tiling_on_sc=False),
    scratch_types=[
        pltpu.VMEM((_BRR,), jnp.int32),
        pltpu.VMEM((_BRR,), jnp.int32),
        pltpu.VMEM((_BRR,), jnp.float32),
        pltpu.VMEM((_BRR,), jnp.float32),
        pltpu.VMEM((_BRR,), jnp.float32),
        pltpu.VMEM((_BRR,), jnp.float32),
        pltpu.VMEM((16,), jnp.int32),
        pltpu.SemaphoreType.DMA,
    ],
)
def _blur1_chain(w0, p0, p1, p2, p3, p4, p5, m0, m1, m2, m3, m4, m5, uarr,
                 wa, wb, ixp, ixm, gp, gm, old, ob, ub, sem):
    wid = lax.axis_index("s")
    pltpu.sync_copy(uarr, ub)
    u = ub[pl.ds(0, 16)][0]
    ps = (p0, p1, p2, p3, p4, p5)
    ms = (m0, m1, m2, m3, m4, m5)
    chain = [(w0, wa), (wa, wb), (wb, wa), (wa, wb), (wb, wa), (wa, wb)]
    for ph in range(_ED):
        src_r, dst_r = chain[ph]
        nbp_, nbm_ = ps[ph], ms[ph]

        @pl.when(wid == 0)
        def _zero_pad():
            ob[pl.ds(0, 16)] = jnp.zeros((16,), jnp.float32)
            pltpu.sync_copy(ob.at[pl.ds(0, 16)], dst_r.at[pl.ds(0, _PAD1)])

        def blk_body(t, carry):
            row0 = (wid + _NW1 * t) * _BRR

            @pl.when(row0 < u)
            def _go():
                pltpu.sync_copy(nbp_.at[pl.ds(row0, _BRR)], ixp)
                pltpu.sync_copy(nbm_.at[pl.ds(row0, _BRR)], ixm)

                def fire(j, c):
                    sl = pl.ds(j * _GB, _GB)
                    pltpu.async_copy(src_r.at[ixp.at[sl]], gp.at[sl], sem)
                    pltpu.async_copy(src_r.at[ixm.at[sl]], gm.at[sl], sem)
                    return c

                lax.fori_loop(0, _BRR // _GB, fire, 0)
                pltpu.sync_copy(src_r.at[pl.ds(_PAD1 + row0, _BRR)], old)
                pltpu.make_async_copy(src_r.at[pl.ds(0, _BRR)], gp, sem).wait()
                pltpu.make_async_copy(src_r.at[pl.ds(0, _BRR)], gm, sem).wait()

                def blk(i, c):
                    s = pl.ds(i * 16, 16)
                    ob[s] = old[s] + 0.5 * (gp[s] + gm[s])
                    return c

                lax.fori_loop(0, _BRR // 16, blk, 0)
                pltpu.sync_copy(ob, dst_r.at[pl.ds(_PAD1 + row0, _BRR)])

            return carry

        lax.fori_loop(0, _M // (_NW1 * _BRR), blk_body, 0)
        plsc.subcore_barrier()


def kernel(x, y):
    simplices, neighbors, uarr = _fit(x, _SIGMAS)
    b = _bary(x, _SIGMAS)
    sims_flat = simplices.reshape(-1)
    simsp1 = sims_flat + _PAD
    nbis = [(neighbors[:, :, dd] + _PAD).reshape(-1) for dd in range(_ED)]
    bpad = jnp.zeros((_N, _C), dtype=jnp.float32).at[:, :_ED].set(b)

    b_flat = b.reshape(-1)
    s1 = jnp.zeros((_M,), dtype=jnp.float32).at[sims_flat].add(b_flat)
    w = jnp.concatenate([jnp.zeros((_PAD1,), jnp.float32), s1])
    nbp = [neighbors[:, 0, dd] + _PAD1 for dd in range(_ED)]
    nbm = [neighbors[:, 1, dd] + _PAD1 for dd in range(_ED)]
    wa, wb = _blur1_chain(w, nbp[0], nbp[1], nbp[2], nbp[3], nbp[4], nbp[5],
                          nbm[0], nbm[1], nbm[2], nbm[3], nbm[4], nbm[5], uarr)
    g1 = _gather1(wb, sims_flat + _PAD1)
    r1 = (b_flat * g1).reshape(_N, _ED).sum(axis=1) * _ALPHA
    norms = (1.0 / jnp.sqrt(r1 + 1e-20))[:, None]

    out = _filter_sc(y * norms, b, sims_flat, simsp1, nbis, bpad, uarr)
    return out * norms
